# Initial kernel scaffold; baseline (speedup 1.0000x reference)
import jax
import jax.numpy as jnp
from jax.experimental import pallas as pl


def kernel(x, edge_index, batch, conv_w0, conv_b0, bn_g0, bn_b0, conv_w1, conv_b1, bn_g1, bn_b1, conv_w2, conv_b2, bn_g2, bn_b2, out_w1, out_b1, out_w2, out_b2):
    raise NotImplementedError("write your pallas kernel here")


# speedup vs baseline: 16.4565x; 16.4565x over previous
import functools

import jax
import jax.numpy as jnp
from jax import lax
from jax.experimental import pallas as pl
from jax.experimental.pallas import tpu as pltpu
from jax.experimental.pallas import tpu_sc as plsc

N = 10000
D = 128
H = 128
E = 320000
NC = 2
NS = 16
NW = NC * NS
CH = 80
NCHUNK = (E // NW) // CH
RB = 624
TAIL_BASE = NS * RB
TAIL = N - TAIL_BASE
BR = 1000
GRID = N // BR
INVS = (1.0 + 1e-5) ** -0.5

_mesh = plsc.VectorSubcoreMesh(core_axis_name="c", subcore_axis_name="s")


@functools.partial(
    pl.kernel,
    out_type=jax.ShapeDtypeStruct((NC, N, 16), jnp.float32),
    mesh=_mesh,
    scratch_types=[
        pltpu.VMEM((NCHUNK, CH), jnp.int32),
        pltpu.VMEM((CH, 16), jnp.float32),
        pltpu.VMEM_SHARED((N, 16), jnp.float32),
    ],
)
def _sc_degree(dst_hbm, ones_hbm, out_hbm, dstv, onesb, acc):
    c = lax.axis_index("c")
    s = lax.axis_index("s")
    wid = c * NS + s
    pltpu.sync_copy(dst_hbm.at[wid], dstv)

    @pl.loop(0, CH)
    def _(i):
        onesb[i] = jnp.ones((16,), jnp.float32)

    pltpu.sync_copy(ones_hbm.at[pl.ds(s * RB, RB)], acc.at[pl.ds(s * RB, RB)])

    @pl.when(s == 0)
    def _():
        pltpu.sync_copy(ones_hbm.at[pl.ds(TAIL_BASE, TAIL)],
                        acc.at[pl.ds(TAIL_BASE, TAIL)])

    plsc.subcore_barrier()

    @pl.loop(0, NCHUNK)
    def _(j):
        pltpu.sync_copy(onesb, acc.at[dstv.at[j]], add=True)

    plsc.subcore_barrier()
    pltpu.sync_copy(acc.at[pl.ds(s * RB, RB)], out_hbm.at[c, pl.ds(s * RB, RB)])

    @pl.when(s == 0)
    def _():
        pltpu.sync_copy(acc.at[pl.ds(TAIL_BASE, TAIL)],
                        out_hbm.at[c, pl.ds(TAIL_BASE, TAIL)])


@functools.partial(
    pl.kernel,
    out_type=jax.ShapeDtypeStruct((NC, N, H), jnp.float32),
    mesh=_mesh,
    scratch_types=[
        pltpu.VMEM((NCHUNK, CH), jnp.int32),
        pltpu.VMEM((NCHUNK, CH), jnp.int32),
        pltpu.VMEM((CH, H), jnp.float32),
        pltpu.VMEM_SHARED((N, H), jnp.float32),
    ],
)
def _sc_gather_scatter(hh_hbm, src_hbm, dst_hbm, out_hbm, srcv, dstv, buf, acc):
    c = lax.axis_index("c")
    s = lax.axis_index("s")
    wid = c * NS + s
    pltpu.sync_copy(src_hbm.at[wid], srcv)
    pltpu.sync_copy(dst_hbm.at[wid], dstv)
    pltpu.sync_copy(hh_hbm.at[pl.ds(s * RB, RB)], acc.at[pl.ds(s * RB, RB)])

    @pl.when(s == 0)
    def _():
        pltpu.sync_copy(hh_hbm.at[pl.ds(TAIL_BASE, TAIL)],
                        acc.at[pl.ds(TAIL_BASE, TAIL)])

    plsc.subcore_barrier()

    @pl.loop(0, NCHUNK)
    def _(j):
        pltpu.sync_copy(hh_hbm.at[srcv.at[j]], buf)
        pltpu.sync_copy(buf, acc.at[dstv.at[j]], add=True)

    plsc.subcore_barrier()
    pltpu.sync_copy(acc.at[pl.ds(s * RB, RB)], out_hbm.at[c, pl.ds(s * RB, RB)])

    @pl.when(s == 0)
    def _():
        pltpu.sync_copy(acc.at[pl.ds(TAIL_BASE, TAIL)],
                        out_hbm.at[c, pl.ds(TAIL_BASE, TAIL)])


def _mm_body(x_ref, w_ref, o_ref):
    o_ref[...] = jnp.dot(x_ref[...], w_ref[...], preferred_element_type=jnp.float32)


def _tc_matmul(x, w):
    return pl.pallas_call(
        _mm_body,
        grid=(GRID,),
        in_specs=[
            pl.BlockSpec((BR, D), lambda i: (i, 0)),
            pl.BlockSpec((D, H), lambda i: (0, 0)),
        ],
        out_specs=pl.BlockSpec((BR, H), lambda i: (i, 0)),
        out_shape=jax.ShapeDtypeStruct((N, H), jnp.float32),
    )(x, w)


def _scale_body(xw_ref, deg_ref, hh_ref, dinv_ref):
    deg = deg_ref[0, :, 0:1] + deg_ref[1, :, 0:1] - 1.0
    dinv = lax.rsqrt(deg)
    dinv_ref[...] = dinv
    hh_ref[...] = xw_ref[...] * dinv


def _tc_scale(xw, degp):
    return pl.pallas_call(
        _scale_body,
        grid=(GRID,),
        in_specs=[
            pl.BlockSpec((BR, H), lambda i: (i, 0)),
            pl.BlockSpec((NC, BR, 16), lambda i: (0, i, 0)),
        ],
        out_specs=[
            pl.BlockSpec((BR, H), lambda i: (i, 0)),
            pl.BlockSpec((BR, 1), lambda i: (i, 0)),
        ],
        out_shape=[
            jax.ShapeDtypeStruct((N, H), jnp.float32),
            jax.ShapeDtypeStruct((N, 1), jnp.float32),
        ],
    )(xw, degp)


def _post_body(p_ref, hh_ref, dinv_ref, b_ref, g_ref, bb_ref, w_ref, o_ref):
    dinv = dinv_ref[...]
    y = (p_ref[0] + p_ref[1] - hh_ref[...]) * dinv + b_ref[...]
    t = jnp.maximum(y * INVS * g_ref[...] + bb_ref[...], 0.0)
    o_ref[...] = jnp.dot(t, w_ref[...], preferred_element_type=jnp.float32) * dinv


def _tc_post(p, hh, dinv, b, g, bb, w_next):
    return pl.pallas_call(
        _post_body,
        grid=(GRID,),
        in_specs=[
            pl.BlockSpec((NC, BR, H), lambda i: (0, i, 0)),
            pl.BlockSpec((BR, H), lambda i: (i, 0)),
            pl.BlockSpec((BR, 1), lambda i: (i, 0)),
            pl.BlockSpec((1, H), lambda i: (0, 0)),
            pl.BlockSpec((1, H), lambda i: (0, 0)),
            pl.BlockSpec((1, H), lambda i: (0, 0)),
            pl.BlockSpec((H, H), lambda i: (0, 0)),
        ],
        out_specs=pl.BlockSpec((BR, H), lambda i: (i, 0)),
        out_shape=jax.ShapeDtypeStruct((N, H), jnp.float32),
    )(p, hh, dinv, b, g, bb, w_next)


def _final_body(p_ref, hh_ref, dinv_ref, b_ref, g_ref, bb_ref,
                w1_ref, b1_ref, w2_ref, b2_ref, o_ref):
    dinv = dinv_ref[...]
    y = (p_ref[0] + p_ref[1] - hh_ref[...]) * dinv + b_ref[...]
    t = jnp.maximum(y * INVS * g_ref[...] + bb_ref[...], 0.0)
    z = jnp.maximum(
        jnp.dot(t, w1_ref[...], preferred_element_type=jnp.float32) + b1_ref[...], 0.0)
    o_ref[...] = jnp.dot(z, w2_ref[...], preferred_element_type=jnp.float32) + b2_ref[...]


def _tc_final(p, hh, dinv, b, g, bb, w1, b1, w2, b2):
    return pl.pallas_call(
        _final_body,
        grid=(GRID,),
        in_specs=[
            pl.BlockSpec((NC, BR, H), lambda i: (0, i, 0)),
            pl.BlockSpec((BR, H), lambda i: (i, 0)),
            pl.BlockSpec((BR, 1), lambda i: (i, 0)),
            pl.BlockSpec((1, H), lambda i: (0, 0)),
            pl.BlockSpec((1, H), lambda i: (0, 0)),
            pl.BlockSpec((1, H), lambda i: (0, 0)),
            pl.BlockSpec((H, H // 2), lambda i: (0, 0)),
            pl.BlockSpec((1, H // 2), lambda i: (0, 0)),
            pl.BlockSpec((H // 2, 2), lambda i: (0, 0)),
            pl.BlockSpec((1, 2), lambda i: (0, 0)),
        ],
        out_specs=pl.BlockSpec((BR, 2), lambda i: (i, 0)),
        out_shape=jax.ShapeDtypeStruct((N, 2), jnp.float32),
    )(p, hh, dinv, b, g, bb, w1, b1, w2, b2)


def kernel(x, edge_index, batch, conv_w0, conv_b0, bn_g0, bn_b0,
           conv_w1, conv_b1, bn_g1, bn_b1, conv_w2, conv_b2, bn_g2, bn_b2,
           out_w1, out_b1, out_w2, out_b2):
    src_r = edge_index[0].reshape(NW, NCHUNK, CH)
    dst_r = edge_index[1].reshape(NW, NCHUNK, CH)
    ones16 = jnp.ones((N, 16), jnp.float32)

    degp = _sc_degree(dst_r, ones16)
    xw = _tc_matmul(x, conv_w0)
    hh, dinv = _tc_scale(xw, degp)

    p = _sc_gather_scatter(hh, src_r, dst_r)
    hh = _tc_post(p, hh, dinv, conv_b0.reshape(1, H), bn_g0.reshape(1, H),
                  bn_b0.reshape(1, H), conv_w1)

    p = _sc_gather_scatter(hh, src_r, dst_r)
    hh = _tc_post(p, hh, dinv, conv_b1.reshape(1, H), bn_g1.reshape(1, H),
                  bn_b1.reshape(1, H), conv_w2)

    p = _sc_gather_scatter(hh, src_r, dst_r)
    return _tc_final(p, hh, dinv, conv_b2.reshape(1, H), bn_g2.reshape(1, H),
                     bn_b2.reshape(1, H), out_w1, out_b1.reshape(1, H // 2),
                     out_w2, out_b2.reshape(1, 2))


# speedup vs baseline: 24.1380x; 1.4668x over previous
import functools

import jax
import jax.numpy as jnp
from jax import lax
from jax.experimental import pallas as pl
from jax.experimental.pallas import tpu as pltpu
from jax.experimental.pallas import tpu_sc as plsc

N = 10000
D = 128
H = 128
E = 320000
NC = 2
NS = 16
NW = NC * NS
CH = 80
NCHUNK = (E // NW) // CH
PCH = 25
NPHASE = NCHUNK // PCH
RB = 624
TAIL_BASE = NS * RB
TAIL = N - TAIL_BASE
BR = 1000
GRID = N // BR
INVS = (1.0 + 1e-5) ** -0.5

_mesh = plsc.VectorSubcoreMesh(core_axis_name="c", subcore_axis_name="s")


@functools.partial(
    pl.kernel,
    out_type=jax.ShapeDtypeStruct((NC, N, 16), jnp.float32),
    mesh=_mesh,
    scratch_types=[
        pltpu.VMEM((NPHASE, PCH, CH), jnp.int32),
        pltpu.VMEM((CH, 16), jnp.float32),
        pltpu.VMEM_SHARED((N, 16), jnp.float32),
    ],
)
def _sc_degree(dst_hbm, ones_hbm, out_hbm, dstv, onesb, acc):
    c = lax.axis_index("c")
    s = lax.axis_index("s")
    wid = c * NS + s
    pltpu.sync_copy(dst_hbm.at[wid], dstv)

    @pl.loop(0, CH)
    def _(i):
        onesb[i] = jnp.ones((16,), jnp.float32)

    pltpu.sync_copy(ones_hbm.at[pl.ds(s * RB, RB)], acc.at[pl.ds(s * RB, RB)])

    @pl.when(s == 0)
    def _():
        pltpu.sync_copy(ones_hbm.at[pl.ds(TAIL_BASE, TAIL)],
                        acc.at[pl.ds(TAIL_BASE, TAIL)])

    plsc.subcore_barrier()

    @pl.loop(0, NPHASE)
    def _(p):
        @pl.loop(0, PCH)
        def _(j):
            pltpu.sync_copy(onesb, acc.at[dstv.at[p, j]], add=True)

    plsc.subcore_barrier()
    pltpu.sync_copy(acc.at[pl.ds(s * RB, RB)], out_hbm.at[c, pl.ds(s * RB, RB)])

    @pl.when(s == 0)
    def _():
        pltpu.sync_copy(acc.at[pl.ds(TAIL_BASE, TAIL)],
                        out_hbm.at[c, pl.ds(TAIL_BASE, TAIL)])


@functools.partial(
    pl.kernel,
    out_type=jax.ShapeDtypeStruct((NC, N, H), jnp.float32),
    mesh=_mesh,
    scratch_types=[
        pltpu.VMEM((PCH, CH), jnp.int32),
        pltpu.VMEM((PCH, CH), jnp.int32),
        pltpu.VMEM((CH, H), jnp.float32),
        pltpu.VMEM((CH, H), jnp.float32),
        pltpu.VMEM_SHARED((N, H), jnp.float32),
        pltpu.SemaphoreType.DMA,
        pltpu.SemaphoreType.DMA,
    ],
)
def _sc_gather_scatter(hh_hbm, src_hbm, dst_hbm, out_hbm, srcv, dstv,
                       buf0, buf1, acc, sem0, sem1):
    c = lax.axis_index("c")
    s = lax.axis_index("s")
    wid = c * NS + s
    pltpu.sync_copy(hh_hbm.at[pl.ds(s * RB, RB)], acc.at[pl.ds(s * RB, RB)])

    @pl.when(s == 0)
    def _():
        pltpu.sync_copy(hh_hbm.at[pl.ds(TAIL_BASE, TAIL)],
                        acc.at[pl.ds(TAIL_BASE, TAIL)])

    plsc.subcore_barrier()

    def gstart(j, buf, sem):
        pltpu.make_async_copy(hh_hbm.at[srcv.at[j]], buf, sem).start()

    def gwait(j, buf, sem):
        pltpu.make_async_copy(hh_hbm.at[srcv.at[j]], buf, sem).wait()

    @pl.loop(0, NPHASE)
    def _(p):
        pltpu.sync_copy(src_hbm.at[wid, p], srcv)
        pltpu.sync_copy(dst_hbm.at[wid, p], dstv)
        gstart(0, buf0, sem0)

        @pl.loop(0, PCH - 1, step=2)
        def _(j):
            gstart(j + 1, buf1, sem1)
            gwait(j, buf0, sem0)
            pltpu.sync_copy(buf0, acc.at[dstv.at[j]], add=True)
            gstart(j + 2, buf0, sem0)
            gwait(j + 1, buf1, sem1)
            pltpu.sync_copy(buf1, acc.at[dstv.at[j + 1]], add=True)

        gwait(PCH - 1, buf0, sem0)
        pltpu.sync_copy(buf0, acc.at[dstv.at[PCH - 1]], add=True)

    plsc.subcore_barrier()
    pltpu.sync_copy(acc.at[pl.ds(s * RB, RB)], out_hbm.at[c, pl.ds(s * RB, RB)])

    @pl.when(s == 0)
    def _():
        pltpu.sync_copy(acc.at[pl.ds(TAIL_BASE, TAIL)],
                        out_hbm.at[c, pl.ds(TAIL_BASE, TAIL)])


def _mm_body(x_ref, w_ref, o_ref):
    o_ref[...] = jnp.dot(x_ref[...], w_ref[...], preferred_element_type=jnp.float32)


def _tc_matmul(x, w):
    return pl.pallas_call(
        _mm_body,
        grid=(GRID,),
        in_specs=[
            pl.BlockSpec((BR, D), lambda i: (i, 0)),
            pl.BlockSpec((D, H), lambda i: (0, 0)),
        ],
        out_specs=pl.BlockSpec((BR, H), lambda i: (i, 0)),
        out_shape=jax.ShapeDtypeStruct((N, H), jnp.float32),
    )(x, w)


def _scale_body(xw_ref, deg_ref, hh_ref, dinv_ref):
    deg = deg_ref[0, :, 0:1] + deg_ref[1, :, 0:1] - 1.0
    dinv = lax.rsqrt(deg)
    dinv_ref[...] = dinv
    hh_ref[...] = xw_ref[...] * dinv


def _tc_scale(xw, degp):
    return pl.pallas_call(
        _scale_body,
        grid=(GRID,),
        in_specs=[
            pl.BlockSpec((BR, H), lambda i: (i, 0)),
            pl.BlockSpec((NC, BR, 16), lambda i: (0, i, 0)),
        ],
        out_specs=[
            pl.BlockSpec((BR, H), lambda i: (i, 0)),
            pl.BlockSpec((BR, 1), lambda i: (i, 0)),
        ],
        out_shape=[
            jax.ShapeDtypeStruct((N, H), jnp.float32),
            jax.ShapeDtypeStruct((N, 1), jnp.float32),
        ],
    )(xw, degp)


def _post_body(p_ref, hh_ref, dinv_ref, b_ref, g_ref, bb_ref, w_ref, o_ref):
    dinv = dinv_ref[...]
    y = (p_ref[0] + p_ref[1] - hh_ref[...]) * dinv + b_ref[...]
    t = jnp.maximum(y * INVS * g_ref[...] + bb_ref[...], 0.0)
    o_ref[...] = jnp.dot(t, w_ref[...], preferred_element_type=jnp.float32) * dinv


def _tc_post(p, hh, dinv, b, g, bb, w_next):
    return pl.pallas_call(
        _post_body,
        grid=(GRID,),
        in_specs=[
            pl.BlockSpec((NC, BR, H), lambda i: (0, i, 0)),
            pl.BlockSpec((BR, H), lambda i: (i, 0)),
            pl.BlockSpec((BR, 1), lambda i: (i, 0)),
            pl.BlockSpec((1, H), lambda i: (0, 0)),
            pl.BlockSpec((1, H), lambda i: (0, 0)),
            pl.BlockSpec((1, H), lambda i: (0, 0)),
            pl.BlockSpec((H, H), lambda i: (0, 0)),
        ],
        out_specs=pl.BlockSpec((BR, H), lambda i: (i, 0)),
        out_shape=jax.ShapeDtypeStruct((N, H), jnp.float32),
    )(p, hh, dinv, b, g, bb, w_next)


def _final_body(p_ref, hh_ref, dinv_ref, b_ref, g_ref, bb_ref,
                w1_ref, b1_ref, w2_ref, b2_ref, o_ref):
    dinv = dinv_ref[...]
    y = (p_ref[0] + p_ref[1] - hh_ref[...]) * dinv + b_ref[...]
    t = jnp.maximum(y * INVS * g_ref[...] + bb_ref[...], 0.0)
    z = jnp.maximum(
        jnp.dot(t, w1_ref[...], preferred_element_type=jnp.float32) + b1_ref[...], 0.0)
    o_ref[...] = jnp.dot(z, w2_ref[...], preferred_element_type=jnp.float32) + b2_ref[...]


def _tc_final(p, hh, dinv, b, g, bb, w1, b1, w2, b2):
    return pl.pallas_call(
        _final_body,
        grid=(GRID,),
        in_specs=[
            pl.BlockSpec((NC, BR, H), lambda i: (0, i, 0)),
            pl.BlockSpec((BR, H), lambda i: (i, 0)),
            pl.BlockSpec((BR, 1), lambda i: (i, 0)),
            pl.BlockSpec((1, H), lambda i: (0, 0)),
            pl.BlockSpec((1, H), lambda i: (0, 0)),
            pl.BlockSpec((1, H), lambda i: (0, 0)),
            pl.BlockSpec((H, H // 2), lambda i: (0, 0)),
            pl.BlockSpec((1, H // 2), lambda i: (0, 0)),
            pl.BlockSpec((H // 2, 2), lambda i: (0, 0)),
            pl.BlockSpec((1, 2), lambda i: (0, 0)),
        ],
        out_specs=pl.BlockSpec((BR, 2), lambda i: (i, 0)),
        out_shape=jax.ShapeDtypeStruct((N, 2), jnp.float32),
    )(p, hh, dinv, b, g, bb, w1, b1, w2, b2)


def kernel(x, edge_index, batch, conv_w0, conv_b0, bn_g0, bn_b0,
           conv_w1, conv_b1, bn_g1, bn_b1, conv_w2, conv_b2, bn_g2, bn_b2,
           out_w1, out_b1, out_w2, out_b2):
    src_r = edge_index[0].reshape(NW, NPHASE, PCH, CH)
    dst_r = edge_index[1].reshape(NW, NPHASE, PCH, CH)
    ones16 = jnp.ones((N, 16), jnp.float32)

    degp = _sc_degree(dst_r, ones16)
    xw = _tc_matmul(x, conv_w0)
    hh, dinv = _tc_scale(xw, degp)

    p = _sc_gather_scatter(hh, src_r, dst_r)
    hh = _tc_post(p, hh, dinv, conv_b0.reshape(1, H), bn_g0.reshape(1, H),
                  bn_b0.reshape(1, H), conv_w1)

    p = _sc_gather_scatter(hh, src_r, dst_r)
    hh = _tc_post(p, hh, dinv, conv_b1.reshape(1, H), bn_g1.reshape(1, H),
                  bn_b1.reshape(1, H), conv_w2)

    p = _sc_gather_scatter(hh, src_r, dst_r)
    return _tc_final(p, hh, dinv, conv_b2.reshape(1, H), bn_g2.reshape(1, H),
                     bn_b2.reshape(1, H), out_w1, out_b1.reshape(1, H // 2),
                     out_w2, out_b2.reshape(1, 2))


# speedup vs baseline: 27.5898x; 1.1430x over previous
import functools

import jax
import jax.numpy as jnp
from jax import lax
from jax.experimental import pallas as pl
from jax.experimental.pallas import tpu as pltpu
from jax.experimental.pallas import tpu_sc as plsc

N = 10000
D = 128
H = 128
E = 320000
NC = 2
NS = 16
NW = NC * NS
CH = 80
NCHUNK = (E // NW) // CH
PCH = 25
NPHASE = NCHUNK // PCH
RB = 624
TAIL_BASE = NS * RB
TAIL = N - TAIL_BASE
BR = 1000
GRID = N // BR
INVS = (1.0 + 1e-5) ** -0.5

_mesh = plsc.VectorSubcoreMesh(core_axis_name="c", subcore_axis_name="s")


@functools.partial(
    pl.kernel,
    out_type=jax.ShapeDtypeStruct((NC, N, 16), jnp.float32),
    mesh=_mesh,
    scratch_types=[
        pltpu.VMEM((NPHASE, PCH, CH), jnp.int32),
        pltpu.VMEM((CH, 16), jnp.float32),
        pltpu.VMEM_SHARED((N, 16), jnp.float32),
        pltpu.SemaphoreType.DMA,
    ],
)
def _sc_degree(dst_hbm, ones_hbm, out_hbm, dstv, onesb, acc, sem):
    c = lax.axis_index("c")
    s = lax.axis_index("s")
    wid = c * NS + s
    pltpu.sync_copy(dst_hbm.at[wid], dstv)

    @pl.loop(0, CH)
    def _(i):
        onesb[i] = jnp.ones((16,), jnp.float32)

    pltpu.sync_copy(ones_hbm.at[pl.ds(s * RB, RB)], acc.at[pl.ds(s * RB, RB)])

    @pl.when(s == 0)
    def _():
        pltpu.sync_copy(ones_hbm.at[pl.ds(TAIL_BASE, TAIL)],
                        acc.at[pl.ds(TAIL_BASE, TAIL)])

    plsc.subcore_barrier()

    @pl.loop(0, NPHASE)
    def _(p):
        @pl.loop(0, PCH)
        def _(j):
            pltpu.async_copy(onesb, acc.at[dstv.at[p, j]], sem, add=True)

    @pl.loop(0, NPHASE * PCH)
    def _(j):
        pltpu.make_async_copy(onesb, acc.at[dstv.at[0, 0]], sem).wait()

    plsc.subcore_barrier()
    pltpu.sync_copy(acc.at[pl.ds(s * RB, RB)], out_hbm.at[c, pl.ds(s * RB, RB)])

    @pl.when(s == 0)
    def _():
        pltpu.sync_copy(acc.at[pl.ds(TAIL_BASE, TAIL)],
                        out_hbm.at[c, pl.ds(TAIL_BASE, TAIL)])


@functools.partial(
    pl.kernel,
    out_type=jax.ShapeDtypeStruct((NC, N, H), jnp.float32),
    mesh=_mesh,
    scratch_types=[
        pltpu.VMEM((PCH, CH), jnp.int32),
        pltpu.VMEM((PCH, CH), jnp.int32),
        pltpu.VMEM((CH, H), jnp.float32),
        pltpu.VMEM((CH, H), jnp.float32),
        pltpu.VMEM((CH, H), jnp.float32),
        pltpu.VMEM_SHARED((N, H), jnp.float32),
        pltpu.SemaphoreType.DMA,
        pltpu.SemaphoreType.DMA,
        pltpu.SemaphoreType.DMA,
    ],
)
def _sc_gather_scatter(hh_hbm, src_hbm, dst_hbm, out_hbm, srcv, dstv,
                       buf0, buf1, buf2, acc, sem0, sem1, sem2):
    c = lax.axis_index("c")
    s = lax.axis_index("s")
    wid = c * NS + s
    pltpu.sync_copy(hh_hbm.at[pl.ds(s * RB, RB)], acc.at[pl.ds(s * RB, RB)])

    @pl.when(s == 0)
    def _():
        pltpu.sync_copy(hh_hbm.at[pl.ds(TAIL_BASE, TAIL)],
                        acc.at[pl.ds(TAIL_BASE, TAIL)])

    plsc.subcore_barrier()

    def gstart(j, buf, sem):
        pltpu.make_async_copy(hh_hbm.at[srcv.at[j]], buf, sem).start()

    def gwait(j, buf, sem):
        pltpu.make_async_copy(hh_hbm.at[srcv.at[j]], buf, sem).wait()

    @pl.loop(0, NPHASE)
    def _(p):
        pltpu.sync_copy(src_hbm.at[wid, p], srcv)
        pltpu.sync_copy(dst_hbm.at[wid, p], dstv)
        gstart(0, buf0, sem0)
        gstart(1, buf1, sem1)

        @pl.loop(0, PCH - 1, step=3)
        def _(j):
            gstart(j + 2, buf2, sem2)
            gwait(j, buf0, sem0)
            pltpu.sync_copy(buf0, acc.at[dstv.at[j]], add=True)
            gstart(j + 3, buf0, sem0)
            gwait(j + 1, buf1, sem1)
            pltpu.sync_copy(buf1, acc.at[dstv.at[j + 1]], add=True)

            @pl.when(j + 4 < PCH)
            def _():
                gstart(j + 4, buf1, sem1)

            gwait(j + 2, buf2, sem2)
            pltpu.sync_copy(buf2, acc.at[dstv.at[j + 2]], add=True)

        gwait(PCH - 1, buf0, sem0)
        pltpu.sync_copy(buf0, acc.at[dstv.at[PCH - 1]], add=True)

    plsc.subcore_barrier()
    pltpu.sync_copy(acc.at[pl.ds(s * RB, RB)], out_hbm.at[c, pl.ds(s * RB, RB)])

    @pl.when(s == 0)
    def _():
        pltpu.sync_copy(acc.at[pl.ds(TAIL_BASE, TAIL)],
                        out_hbm.at[c, pl.ds(TAIL_BASE, TAIL)])


def _mm_body(x_ref, w_ref, o_ref):
    o_ref[...] = jnp.dot(x_ref[...], w_ref[...], preferred_element_type=jnp.float32)


def _tc_matmul(x, w):
    return pl.pallas_call(
        _mm_body,
        grid=(GRID,),
        in_specs=[
            pl.BlockSpec((BR, D), lambda i: (i, 0)),
            pl.BlockSpec((D, H), lambda i: (0, 0)),
        ],
        out_specs=pl.BlockSpec((BR, H), lambda i: (i, 0)),
        out_shape=jax.ShapeDtypeStruct((N, H), jnp.float32),
    )(x, w)


def _scale_body(xw_ref, deg_ref, hh_ref, dinv_ref):
    deg = deg_ref[0, :, 0:1] + deg_ref[1, :, 0:1] - 1.0
    dinv = lax.rsqrt(deg)
    dinv_ref[...] = dinv
    hh_ref[...] = xw_ref[...] * dinv


def _tc_scale(xw, degp):
    return pl.pallas_call(
        _scale_body,
        grid=(GRID,),
        in_specs=[
            pl.BlockSpec((BR, H), lambda i: (i, 0)),
            pl.BlockSpec((NC, BR, 16), lambda i: (0, i, 0)),
        ],
        out_specs=[
            pl.BlockSpec((BR, H), lambda i: (i, 0)),
            pl.BlockSpec((BR, 1), lambda i: (i, 0)),
        ],
        out_shape=[
            jax.ShapeDtypeStruct((N, H), jnp.float32),
            jax.ShapeDtypeStruct((N, 1), jnp.float32),
        ],
    )(xw, degp)


def _post_body(p_ref, hh_ref, dinv_ref, b_ref, g_ref, bb_ref, w_ref, o_ref):
    dinv = dinv_ref[...]
    y = (p_ref[0] + p_ref[1] - hh_ref[...]) * dinv + b_ref[...]
    t = jnp.maximum(y * INVS * g_ref[...] + bb_ref[...], 0.0)
    o_ref[...] = jnp.dot(t, w_ref[...], preferred_element_type=jnp.float32) * dinv


def _tc_post(p, hh, dinv, b, g, bb, w_next):
    return pl.pallas_call(
        _post_body,
        grid=(GRID,),
        in_specs=[
            pl.BlockSpec((NC, BR, H), lambda i: (0, i, 0)),
            pl.BlockSpec((BR, H), lambda i: (i, 0)),
            pl.BlockSpec((BR, 1), lambda i: (i, 0)),
            pl.BlockSpec((1, H), lambda i: (0, 0)),
            pl.BlockSpec((1, H), lambda i: (0, 0)),
            pl.BlockSpec((1, H), lambda i: (0, 0)),
            pl.BlockSpec((H, H), lambda i: (0, 0)),
        ],
        out_specs=pl.BlockSpec((BR, H), lambda i: (i, 0)),
        out_shape=jax.ShapeDtypeStruct((N, H), jnp.float32),
    )(p, hh, dinv, b, g, bb, w_next)


def _final_body(p_ref, hh_ref, dinv_ref, b_ref, g_ref, bb_ref,
                w1_ref, b1_ref, w2_ref, b2_ref, o_ref):
    dinv = dinv_ref[...]
    y = (p_ref[0] + p_ref[1] - hh_ref[...]) * dinv + b_ref[...]
    t = jnp.maximum(y * INVS * g_ref[...] + bb_ref[...], 0.0)
    z = jnp.maximum(
        jnp.dot(t, w1_ref[...], preferred_element_type=jnp.float32) + b1_ref[...], 0.0)
    o_ref[...] = jnp.dot(z, w2_ref[...], preferred_element_type=jnp.float32) + b2_ref[...]


def _tc_final(p, hh, dinv, b, g, bb, w1, b1, w2, b2):
    return pl.pallas_call(
        _final_body,
        grid=(GRID,),
        in_specs=[
            pl.BlockSpec((NC, BR, H), lambda i: (0, i, 0)),
            pl.BlockSpec((BR, H), lambda i: (i, 0)),
            pl.BlockSpec((BR, 1), lambda i: (i, 0)),
            pl.BlockSpec((1, H), lambda i: (0, 0)),
            pl.BlockSpec((1, H), lambda i: (0, 0)),
            pl.BlockSpec((1, H), lambda i: (0, 0)),
            pl.BlockSpec((H, H // 2), lambda i: (0, 0)),
            pl.BlockSpec((1, H // 2), lambda i: (0, 0)),
            pl.BlockSpec((H // 2, 2), lambda i: (0, 0)),
            pl.BlockSpec((1, 2), lambda i: (0, 0)),
        ],
        out_specs=pl.BlockSpec((BR, 2), lambda i: (i, 0)),
        out_shape=jax.ShapeDtypeStruct((N, 2), jnp.float32),
    )(p, hh, dinv, b, g, bb, w1, b1, w2, b2)


def kernel(x, edge_index, batch, conv_w0, conv_b0, bn_g0, bn_b0,
           conv_w1, conv_b1, bn_g1, bn_b1, conv_w2, conv_b2, bn_g2, bn_b2,
           out_w1, out_b1, out_w2, out_b2):
    src_r = edge_index[0].reshape(NW, NPHASE, PCH, CH)
    dst_r = edge_index[1].reshape(NW, NPHASE, PCH, CH)
    ones16 = jnp.ones((N, 16), jnp.float32)

    degp = _sc_degree(dst_r, ones16)
    xw = _tc_matmul(x, conv_w0)
    hh, dinv = _tc_scale(xw, degp)

    p = _sc_gather_scatter(hh, src_r, dst_r)
    hh = _tc_post(p, hh, dinv, conv_b0.reshape(1, H), bn_g0.reshape(1, H),
                  bn_b0.reshape(1, H), conv_w1)

    p = _sc_gather_scatter(hh, src_r, dst_r)
    hh = _tc_post(p, hh, dinv, conv_b1.reshape(1, H), bn_g1.reshape(1, H),
                  bn_b1.reshape(1, H), conv_w2)

    p = _sc_gather_scatter(hh, src_r, dst_r)
    return _tc_final(p, hh, dinv, conv_b2.reshape(1, H), bn_g2.reshape(1, H),
                     bn_b2.reshape(1, H), out_w1, out_b1.reshape(1, H // 2),
                     out_w2, out_b2.reshape(1, 2))


# speedup vs baseline: 27.6987x; 1.0039x over previous
import functools

import jax
import jax.numpy as jnp
from jax import lax
from jax.experimental import pallas as pl
from jax.experimental.pallas import tpu as pltpu
from jax.experimental.pallas import tpu_sc as plsc

N = 10000
D = 128
H = 128
E = 320000
NC = 2
NS = 16
NW = NC * NS
CH = 80
NCHUNK = (E // NW) // CH
PCH = 25
NPHASE = NCHUNK // PCH
RB = 624
TAIL_BASE = NS * RB
TAIL = N - TAIL_BASE
BR = 1000
GRID = N // BR
INVS = (1.0 + 1e-5) ** -0.5

_mesh = plsc.VectorSubcoreMesh(core_axis_name="c", subcore_axis_name="s")


@functools.partial(
    pl.kernel,
    out_type=jax.ShapeDtypeStruct((NC, N, 16), jnp.float32),
    mesh=_mesh,
    scratch_types=[
        pltpu.VMEM((NPHASE, PCH, CH), jnp.int32),
        pltpu.VMEM((CH, 16), jnp.float32),
        pltpu.VMEM_SHARED((N, 16), jnp.float32),
        pltpu.SemaphoreType.DMA,
    ],
)
def _sc_degree(dst_hbm, ones_hbm, out_hbm, dstv, onesb, acc, sem):
    c = lax.axis_index("c")
    s = lax.axis_index("s")
    wid = c * NS + s
    pltpu.sync_copy(dst_hbm.at[wid], dstv)

    @pl.loop(0, CH)
    def _(i):
        onesb[i] = jnp.ones((16,), jnp.float32)

    pltpu.sync_copy(ones_hbm.at[pl.ds(s * RB, RB)], acc.at[pl.ds(s * RB, RB)])

    @pl.when(s == 0)
    def _():
        pltpu.sync_copy(ones_hbm.at[pl.ds(TAIL_BASE, TAIL)],
                        acc.at[pl.ds(TAIL_BASE, TAIL)])

    plsc.subcore_barrier()

    @pl.loop(0, NPHASE)
    def _(p):
        @pl.loop(0, PCH)
        def _(j):
            pltpu.async_copy(onesb, acc.at[dstv.at[p, j]], sem, add=True)

    @pl.loop(0, NPHASE * PCH)
    def _(j):
        pltpu.make_async_copy(onesb, acc.at[dstv.at[0, 0]], sem).wait()

    plsc.subcore_barrier()
    pltpu.sync_copy(acc.at[pl.ds(s * RB, RB)], out_hbm.at[c, pl.ds(s * RB, RB)])

    @pl.when(s == 0)
    def _():
        pltpu.sync_copy(acc.at[pl.ds(TAIL_BASE, TAIL)],
                        out_hbm.at[c, pl.ds(TAIL_BASE, TAIL)])


@functools.partial(
    pl.kernel,
    out_type=jax.ShapeDtypeStruct((NC, N, H), jnp.float32),
    mesh=_mesh,
    scratch_types=[
        pltpu.VMEM((PCH, CH), jnp.int32),
        pltpu.VMEM((PCH, CH), jnp.int32),
        pltpu.VMEM((CH, H), jnp.float32),
        pltpu.VMEM((CH, H), jnp.float32),
        pltpu.VMEM((CH, H), jnp.float32),
        pltpu.VMEM((CH, H), jnp.float32),
        pltpu.VMEM_SHARED((N, H), jnp.float32),
        pltpu.SemaphoreType.DMA,
        pltpu.SemaphoreType.DMA,
        pltpu.SemaphoreType.DMA,
        pltpu.SemaphoreType.DMA,
    ],
)
def _sc_gather_scatter(hh_hbm, src_hbm, dst_hbm, out_hbm, srcv, dstv,
                       buf0, buf1, buf2, buf3, acc,
                       sem0, sem1, sem2, sem3):
    c = lax.axis_index("c")
    s = lax.axis_index("s")
    wid = c * NS + s
    pltpu.sync_copy(hh_hbm.at[pl.ds(s * RB, RB)], acc.at[pl.ds(s * RB, RB)])

    @pl.when(s == 0)
    def _():
        pltpu.sync_copy(hh_hbm.at[pl.ds(TAIL_BASE, TAIL)],
                        acc.at[pl.ds(TAIL_BASE, TAIL)])

    plsc.subcore_barrier()

    def gstart(j, buf, sem):
        pltpu.make_async_copy(hh_hbm.at[srcv.at[j]], buf, sem).start()

    def gwait(j, buf, sem):
        pltpu.make_async_copy(hh_hbm.at[srcv.at[j]], buf, sem).wait()

    @pl.loop(0, NPHASE)
    def _(p):
        pltpu.sync_copy(src_hbm.at[wid, p], srcv)
        pltpu.sync_copy(dst_hbm.at[wid, p], dstv)

        bufs = (buf0, buf1, buf2, buf3)
        sems = (sem0, sem1, sem2, sem3)
        for k in range(3):
            gstart(k, bufs[k], sems[k])

        @pl.loop(0, PCH - 1, step=4)
        def _(j):
            for k in range(4):
                c = j + k

                @pl.when(c + 3 < PCH)
                def _():
                    gstart(c + 3, bufs[(k + 3) % 4], sems[(k + 3) % 4])

                gwait(c, bufs[k], sems[k])
                pltpu.sync_copy(bufs[k], acc.at[dstv.at[c]], add=True)

        gwait(PCH - 1, buf0, sem0)
        pltpu.sync_copy(buf0, acc.at[dstv.at[PCH - 1]], add=True)

    plsc.subcore_barrier()
    pltpu.sync_copy(acc.at[pl.ds(s * RB, RB)], out_hbm.at[c, pl.ds(s * RB, RB)])

    @pl.when(s == 0)
    def _():
        pltpu.sync_copy(acc.at[pl.ds(TAIL_BASE, TAIL)],
                        out_hbm.at[c, pl.ds(TAIL_BASE, TAIL)])


def _mm_body(x_ref, w_ref, o_ref):
    o_ref[...] = jnp.dot(x_ref[...], w_ref[...], preferred_element_type=jnp.float32)


def _tc_matmul(x, w):
    return pl.pallas_call(
        _mm_body,
        grid=(GRID,),
        in_specs=[
            pl.BlockSpec((BR, D), lambda i: (i, 0)),
            pl.BlockSpec((D, H), lambda i: (0, 0)),
        ],
        out_specs=pl.BlockSpec((BR, H), lambda i: (i, 0)),
        out_shape=jax.ShapeDtypeStruct((N, H), jnp.float32),
    )(x, w)


def _scale_body(xw_ref, deg_ref, hh_ref, dinv_ref):
    deg = deg_ref[0, :, 0:1] + deg_ref[1, :, 0:1] - 1.0
    dinv = lax.rsqrt(deg)
    dinv_ref[...] = dinv
    hh_ref[...] = xw_ref[...] * dinv


def _tc_scale(xw, degp):
    return pl.pallas_call(
        _scale_body,
        grid=(GRID,),
        in_specs=[
            pl.BlockSpec((BR, H), lambda i: (i, 0)),
            pl.BlockSpec((NC, BR, 16), lambda i: (0, i, 0)),
        ],
        out_specs=[
            pl.BlockSpec((BR, H), lambda i: (i, 0)),
            pl.BlockSpec((BR, 1), lambda i: (i, 0)),
        ],
        out_shape=[
            jax.ShapeDtypeStruct((N, H), jnp.float32),
            jax.ShapeDtypeStruct((N, 1), jnp.float32),
        ],
    )(xw, degp)


def _post_body(p_ref, hh_ref, dinv_ref, b_ref, g_ref, bb_ref, w_ref, o_ref):
    dinv = dinv_ref[...]
    y = (p_ref[0] + p_ref[1] - hh_ref[...]) * dinv + b_ref[...]
    t = jnp.maximum(y * INVS * g_ref[...] + bb_ref[...], 0.0)
    o_ref[...] = jnp.dot(t, w_ref[...], preferred_element_type=jnp.float32) * dinv


def _tc_post(p, hh, dinv, b, g, bb, w_next):
    return pl.pallas_call(
        _post_body,
        grid=(GRID,),
        in_specs=[
            pl.BlockSpec((NC, BR, H), lambda i: (0, i, 0)),
            pl.BlockSpec((BR, H), lambda i: (i, 0)),
            pl.BlockSpec((BR, 1), lambda i: (i, 0)),
            pl.BlockSpec((1, H), lambda i: (0, 0)),
            pl.BlockSpec((1, H), lambda i: (0, 0)),
            pl.BlockSpec((1, H), lambda i: (0, 0)),
            pl.BlockSpec((H, H), lambda i: (0, 0)),
        ],
        out_specs=pl.BlockSpec((BR, H), lambda i: (i, 0)),
        out_shape=jax.ShapeDtypeStruct((N, H), jnp.float32),
    )(p, hh, dinv, b, g, bb, w_next)


def _final_body(p_ref, hh_ref, dinv_ref, b_ref, g_ref, bb_ref,
                w1_ref, b1_ref, w2_ref, b2_ref, o_ref):
    dinv = dinv_ref[...]
    y = (p_ref[0] + p_ref[1] - hh_ref[...]) * dinv + b_ref[...]
    t = jnp.maximum(y * INVS * g_ref[...] + bb_ref[...], 0.0)
    z = jnp.maximum(
        jnp.dot(t, w1_ref[...], preferred_element_type=jnp.float32) + b1_ref[...], 0.0)
    o_ref[...] = jnp.dot(z, w2_ref[...], preferred_element_type=jnp.float32) + b2_ref[...]


def _tc_final(p, hh, dinv, b, g, bb, w1, b1, w2, b2):
    return pl.pallas_call(
        _final_body,
        grid=(GRID,),
        in_specs=[
            pl.BlockSpec((NC, BR, H), lambda i: (0, i, 0)),
            pl.BlockSpec((BR, H), lambda i: (i, 0)),
            pl.BlockSpec((BR, 1), lambda i: (i, 0)),
            pl.BlockSpec((1, H), lambda i: (0, 0)),
            pl.BlockSpec((1, H), lambda i: (0, 0)),
            pl.BlockSpec((1, H), lambda i: (0, 0)),
            pl.BlockSpec((H, H // 2), lambda i: (0, 0)),
            pl.BlockSpec((1, H // 2), lambda i: (0, 0)),
            pl.BlockSpec((H // 2, 2), lambda i: (0, 0)),
            pl.BlockSpec((1, 2), lambda i: (0, 0)),
        ],
        out_specs=pl.BlockSpec((BR, 2), lambda i: (i, 0)),
        out_shape=jax.ShapeDtypeStruct((N, 2), jnp.float32),
    )(p, hh, dinv, b, g, bb, w1, b1, w2, b2)


def kernel(x, edge_index, batch, conv_w0, conv_b0, bn_g0, bn_b0,
           conv_w1, conv_b1, bn_g1, bn_b1, conv_w2, conv_b2, bn_g2, bn_b2,
           out_w1, out_b1, out_w2, out_b2):
    src_r = edge_index[0].reshape(NW, NPHASE, PCH, CH)
    dst_r = edge_index[1].reshape(NW, NPHASE, PCH, CH)
    ones16 = jnp.ones((N, 16), jnp.float32)

    degp = _sc_degree(dst_r, ones16)
    xw = _tc_matmul(x, conv_w0)
    hh, dinv = _tc_scale(xw, degp)

    p = _sc_gather_scatter(hh, src_r, dst_r)
    hh = _tc_post(p, hh, dinv, conv_b0.reshape(1, H), bn_g0.reshape(1, H),
                  bn_b0.reshape(1, H), conv_w1)

    p = _sc_gather_scatter(hh, src_r, dst_r)
    hh = _tc_post(p, hh, dinv, conv_b1.reshape(1, H), bn_g1.reshape(1, H),
                  bn_b1.reshape(1, H), conv_w2)

    p = _sc_gather_scatter(hh, src_r, dst_r)
    return _tc_final(p, hh, dinv, conv_b2.reshape(1, H), bn_g2.reshape(1, H),
                     bn_b2.reshape(1, H), out_w1, out_b1.reshape(1, H // 2),
                     out_w2, out_b2.reshape(1, 2))


# speedup vs baseline: 27.7230x; 1.0009x over previous
import functools

import jax
import jax.numpy as jnp
from jax import lax
from jax.experimental import pallas as pl
from jax.experimental.pallas import tpu as pltpu
from jax.experimental.pallas import tpu_sc as plsc

N = 10000
D = 128
H = 128
E = 320000
NC = 2
NS = 16
NW = NC * NS
CH = 80
NCHUNK = (E // NW) // CH
PCH = 25
NPHASE = NCHUNK // PCH
RB = 624
TAIL_BASE = NS * RB
TAIL = N - TAIL_BASE
BR = 1000
GRID = N // BR
INVS = (1.0 + 1e-5) ** -0.5

_mesh = plsc.VectorSubcoreMesh(core_axis_name="c", subcore_axis_name="s")


@functools.partial(
    pl.kernel,
    out_type=jax.ShapeDtypeStruct((NC, N, 16), jnp.float32),
    mesh=_mesh,
    scratch_types=[
        pltpu.VMEM((NPHASE, PCH, CH), jnp.int32),
        pltpu.VMEM((CH, 16), jnp.float32),
        pltpu.VMEM_SHARED((N, 16), jnp.float32),
        pltpu.SemaphoreType.DMA,
    ],
)
def _sc_degree(dst_hbm, ones_hbm, out_hbm, dstv, onesb, acc, sem):
    c = lax.axis_index("c")
    s = lax.axis_index("s")
    wid = c * NS + s
    pltpu.sync_copy(dst_hbm.at[wid], dstv)

    @pl.loop(0, CH)
    def _(i):
        onesb[i] = jnp.ones((16,), jnp.float32)

    pltpu.sync_copy(ones_hbm.at[pl.ds(s * RB, RB)], acc.at[pl.ds(s * RB, RB)])

    @pl.when(s == 0)
    def _():
        pltpu.sync_copy(ones_hbm.at[pl.ds(TAIL_BASE, TAIL)],
                        acc.at[pl.ds(TAIL_BASE, TAIL)])

    plsc.subcore_barrier()

    @pl.loop(0, NPHASE)
    def _(p):
        @pl.loop(0, PCH)
        def _(j):
            pltpu.async_copy(onesb, acc.at[dstv.at[p, j]], sem, add=True)

    @pl.loop(0, NPHASE * PCH)
    def _(j):
        pltpu.make_async_copy(onesb, acc.at[dstv.at[0, 0]], sem).wait()

    plsc.subcore_barrier()
    pltpu.sync_copy(acc.at[pl.ds(s * RB, RB)], out_hbm.at[c, pl.ds(s * RB, RB)])

    @pl.when(s == 0)
    def _():
        pltpu.sync_copy(acc.at[pl.ds(TAIL_BASE, TAIL)],
                        out_hbm.at[c, pl.ds(TAIL_BASE, TAIL)])


@functools.partial(
    pl.kernel,
    out_type=jax.ShapeDtypeStruct((NC, N, H), jnp.float32),
    mesh=_mesh,
    scratch_types=[
        pltpu.VMEM((PCH, CH), jnp.int32),
        pltpu.VMEM((PCH, CH), jnp.int32),
        pltpu.VMEM((CH, H), jnp.float32),
        pltpu.VMEM((CH, H), jnp.float32),
        pltpu.VMEM((CH, H), jnp.float32),
        pltpu.VMEM((CH, H), jnp.float32),
        pltpu.VMEM_SHARED((N, H), jnp.float32),
        pltpu.SemaphoreType.DMA,
        pltpu.SemaphoreType.DMA,
        pltpu.SemaphoreType.DMA,
        pltpu.SemaphoreType.DMA,
    ],
)
def _sc_gather_scatter(hh_hbm, src_hbm, dst_hbm, out_hbm, srcv, dstv,
                       buf0, buf1, buf2, buf3, acc,
                       sem0, sem1, sem2, sem3):
    c = lax.axis_index("c")
    s = lax.axis_index("s")
    wid = c * NS + s
    pltpu.sync_copy(hh_hbm.at[pl.ds(s * RB, RB)], acc.at[pl.ds(s * RB, RB)])

    @pl.when(s == 0)
    def _():
        pltpu.sync_copy(hh_hbm.at[pl.ds(TAIL_BASE, TAIL)],
                        acc.at[pl.ds(TAIL_BASE, TAIL)])

    plsc.subcore_barrier()

    def gstart(j, buf, sem):
        pltpu.make_async_copy(hh_hbm.at[srcv.at[j]], buf, sem).start()

    def gwait(j, buf, sem):
        pltpu.make_async_copy(hh_hbm.at[srcv.at[j]], buf, sem).wait()

    @pl.loop(0, NPHASE)
    def _(p):
        pltpu.sync_copy(src_hbm.at[wid, p], srcv)
        pltpu.sync_copy(dst_hbm.at[wid, p], dstv)

        bufs = (buf0, buf1, buf2, buf3)
        sems = (sem0, sem1, sem2, sem3)
        for k in range(3):
            gstart(k, bufs[k], sems[k])

        @pl.loop(0, PCH - 1, step=4)
        def _(j):
            for k in range(4):
                c = j + k

                @pl.when(c + 3 < PCH)
                def _():
                    gstart(c + 3, bufs[(k + 3) % 4], sems[(k + 3) % 4])

                gwait(c, bufs[k], sems[k])
                pltpu.sync_copy(bufs[k], acc.at[dstv.at[c]], add=True)

        gwait(PCH - 1, buf0, sem0)
        pltpu.sync_copy(buf0, acc.at[dstv.at[PCH - 1]], add=True)

    plsc.subcore_barrier()
    pltpu.sync_copy(acc.at[pl.ds(s * RB, RB)], out_hbm.at[c, pl.ds(s * RB, RB)])

    @pl.when(s == 0)
    def _():
        pltpu.sync_copy(acc.at[pl.ds(TAIL_BASE, TAIL)],
                        out_hbm.at[c, pl.ds(TAIL_BASE, TAIL)])


def _scale_body(x_ref, w_ref, deg_ref, hh_ref, dinv_ref):
    deg = deg_ref[0, :, 0:1] + deg_ref[1, :, 0:1] - 1.0
    dinv = lax.rsqrt(deg)
    dinv_ref[...] = dinv
    xw = jnp.dot(x_ref[...], w_ref[...], preferred_element_type=jnp.float32)
    hh_ref[...] = xw * dinv


def _tc_scale(x, w, degp):
    return pl.pallas_call(
        _scale_body,
        grid=(GRID,),
        in_specs=[
            pl.BlockSpec((BR, D), lambda i: (i, 0)),
            pl.BlockSpec((D, H), lambda i: (0, 0)),
            pl.BlockSpec((NC, BR, 16), lambda i: (0, i, 0)),
        ],
        out_specs=[
            pl.BlockSpec((BR, H), lambda i: (i, 0)),
            pl.BlockSpec((BR, 1), lambda i: (i, 0)),
        ],
        out_shape=[
            jax.ShapeDtypeStruct((N, H), jnp.float32),
            jax.ShapeDtypeStruct((N, 1), jnp.float32),
        ],
    )(x, w, degp)


def _post_body(p_ref, hh_ref, dinv_ref, b_ref, g_ref, bb_ref, w_ref, o_ref):
    dinv = dinv_ref[...]
    y = (p_ref[0] + p_ref[1] - hh_ref[...]) * dinv + b_ref[...]
    t = jnp.maximum(y * INVS * g_ref[...] + bb_ref[...], 0.0)
    o_ref[...] = jnp.dot(t, w_ref[...], preferred_element_type=jnp.float32) * dinv


def _tc_post(p, hh, dinv, b, g, bb, w_next):
    return pl.pallas_call(
        _post_body,
        grid=(GRID,),
        in_specs=[
            pl.BlockSpec((NC, BR, H), lambda i: (0, i, 0)),
            pl.BlockSpec((BR, H), lambda i: (i, 0)),
            pl.BlockSpec((BR, 1), lambda i: (i, 0)),
            pl.BlockSpec((1, H), lambda i: (0, 0)),
            pl.BlockSpec((1, H), lambda i: (0, 0)),
            pl.BlockSpec((1, H), lambda i: (0, 0)),
            pl.BlockSpec((H, H), lambda i: (0, 0)),
        ],
        out_specs=pl.BlockSpec((BR, H), lambda i: (i, 0)),
        out_shape=jax.ShapeDtypeStruct((N, H), jnp.float32),
    )(p, hh, dinv, b, g, bb, w_next)


def _final_body(p_ref, hh_ref, dinv_ref, b_ref, g_ref, bb_ref,
                w1_ref, b1_ref, w2_ref, b2_ref, o_ref):
    dinv = dinv_ref[...]
    y = (p_ref[0] + p_ref[1] - hh_ref[...]) * dinv + b_ref[...]
    t = jnp.maximum(y * INVS * g_ref[...] + bb_ref[...], 0.0)
    z = jnp.maximum(
        jnp.dot(t, w1_ref[...], preferred_element_type=jnp.float32) + b1_ref[...], 0.0)
    o_ref[...] = jnp.dot(z, w2_ref[...], preferred_element_type=jnp.float32) + b2_ref[...]


def _tc_final(p, hh, dinv, b, g, bb, w1, b1, w2, b2):
    return pl.pallas_call(
        _final_body,
        grid=(GRID,),
        in_specs=[
            pl.BlockSpec((NC, BR, H), lambda i: (0, i, 0)),
            pl.BlockSpec((BR, H), lambda i: (i, 0)),
            pl.BlockSpec((BR, 1), lambda i: (i, 0)),
            pl.BlockSpec((1, H), lambda i: (0, 0)),
            pl.BlockSpec((1, H), lambda i: (0, 0)),
            pl.BlockSpec((1, H), lambda i: (0, 0)),
            pl.BlockSpec((H, H // 2), lambda i: (0, 0)),
            pl.BlockSpec((1, H // 2), lambda i: (0, 0)),
            pl.BlockSpec((H // 2, 2), lambda i: (0, 0)),
            pl.BlockSpec((1, 2), lambda i: (0, 0)),
        ],
        out_specs=pl.BlockSpec((BR, 2), lambda i: (i, 0)),
        out_shape=jax.ShapeDtypeStruct((N, 2), jnp.float32),
    )(p, hh, dinv, b, g, bb, w1, b1, w2, b2)


def kernel(x, edge_index, batch, conv_w0, conv_b0, bn_g0, bn_b0,
           conv_w1, conv_b1, bn_g1, bn_b1, conv_w2, conv_b2, bn_g2, bn_b2,
           out_w1, out_b1, out_w2, out_b2):
    src_r = edge_index[0].reshape(NW, NPHASE, PCH, CH)
    dst_r = edge_index[1].reshape(NW, NPHASE, PCH, CH)
    ones16 = jnp.ones((N, 16), jnp.float32)

    degp = _sc_degree(dst_r, ones16)
    hh, dinv = _tc_scale(x, conv_w0, degp)

    p = _sc_gather_scatter(hh, src_r, dst_r)
    hh = _tc_post(p, hh, dinv, conv_b0.reshape(1, H), bn_g0.reshape(1, H),
                  bn_b0.reshape(1, H), conv_w1)

    p = _sc_gather_scatter(hh, src_r, dst_r)
    hh = _tc_post(p, hh, dinv, conv_b1.reshape(1, H), bn_g1.reshape(1, H),
                  bn_b1.reshape(1, H), conv_w2)

    p = _sc_gather_scatter(hh, src_r, dst_r)
    return _tc_final(p, hh, dinv, conv_b2.reshape(1, H), bn_g2.reshape(1, H),
                     bn_b2.reshape(1, H), out_w1, out_b1.reshape(1, H // 2),
                     out_w2, out_b2.reshape(1, 2))


# speedup vs baseline: 27.8958x; 1.0062x over previous
import functools

import jax
import jax.numpy as jnp
from jax import lax
from jax.experimental import pallas as pl
from jax.experimental.pallas import tpu as pltpu
from jax.experimental.pallas import tpu_sc as plsc

N = 10000
D = 128
H = 128
E = 320000
NC = 2
NS = 16
NW = NC * NS
EPW = E // NW
CH = 80
NCHUNK = EPW // CH
PCH = 25
PHE = PCH * CH
NPHASE = NCHUNK // PCH
RB = 624
TAIL_BASE = NS * RB
TAIL = N - TAIL_BASE
BR = 2000
GRID = N // BR
INVS = (1.0 + 1e-5) ** -0.5

_mesh = plsc.VectorSubcoreMesh(core_axis_name="c", subcore_axis_name="s")


def _repack(flat, rows2d):
    @pl.loop(0, PCH)
    def _(r):
        @pl.loop(0, CH // 16)
        def _(g):
            rows2d[r, pl.ds(g * 16, 16)] = flat[pl.ds(r * CH + g * 16, 16)]


@functools.partial(
    pl.kernel,
    out_type=jax.ShapeDtypeStruct((NC, N, 16), jnp.float32),
    mesh=_mesh,
    scratch_types=[
        pltpu.VMEM((PHE,), jnp.int32),
        pltpu.VMEM((PCH, CH), jnp.int32),
        pltpu.VMEM((CH, 16), jnp.float32),
        pltpu.VMEM_SHARED((N, 16), jnp.float32),
        pltpu.SemaphoreType.DMA,
    ],
)
def _sc_degree(dst_hbm, ones_hbm, out_hbm, dflat, dstv, onesb, acc, sem):
    c = lax.axis_index("c")
    s = lax.axis_index("s")
    wid = c * NS + s

    @pl.loop(0, CH)
    def _(i):
        onesb[i] = jnp.ones((16,), jnp.float32)

    pltpu.sync_copy(ones_hbm.at[pl.ds(s * RB, RB)], acc.at[pl.ds(s * RB, RB)])

    @pl.when(s == 0)
    def _():
        pltpu.sync_copy(ones_hbm.at[pl.ds(TAIL_BASE, TAIL)],
                        acc.at[pl.ds(TAIL_BASE, TAIL)])

    plsc.subcore_barrier()

    @pl.loop(0, NPHASE)
    def _(p):
        pltpu.sync_copy(dst_hbm.at[pl.ds(wid * EPW + p * PHE, PHE)], dflat)
        _repack(dflat, dstv)

        @pl.loop(0, PCH)
        def _(j):
            pltpu.async_copy(onesb, acc.at[dstv.at[j]], sem, add=True)

        @pl.loop(0, PCH)
        def _(j):
            pltpu.make_async_copy(onesb, acc.at[dstv.at[0]], sem).wait()

    plsc.subcore_barrier()
    pltpu.sync_copy(acc.at[pl.ds(s * RB, RB)], out_hbm.at[c, pl.ds(s * RB, RB)])

    @pl.when(s == 0)
    def _():
        pltpu.sync_copy(acc.at[pl.ds(TAIL_BASE, TAIL)],
                        out_hbm.at[c, pl.ds(TAIL_BASE, TAIL)])


@functools.partial(
    pl.kernel,
    out_type=jax.ShapeDtypeStruct((NC, N, H), jnp.float32),
    mesh=_mesh,
    scratch_types=[
        pltpu.VMEM((PHE,), jnp.int32),
        pltpu.VMEM((PHE,), jnp.int32),
        pltpu.VMEM((PCH, CH), jnp.int32),
        pltpu.VMEM((CH, H), jnp.float32),
        pltpu.VMEM((CH, H), jnp.float32),
        pltpu.VMEM((CH, H), jnp.float32),
        pltpu.VMEM((CH, H), jnp.float32),
        pltpu.VMEM_SHARED((N, H), jnp.float32),
        pltpu.SemaphoreType.DMA,
        pltpu.SemaphoreType.DMA,
        pltpu.SemaphoreType.DMA,
        pltpu.SemaphoreType.DMA,
    ],
)
def _sc_gather_scatter(hh_hbm, src_hbm, dst_hbm, out_hbm, sflat, dflat, dstv,
                       buf0, buf1, buf2, buf3, acc,
                       sem0, sem1, sem2, sem3):
    c = lax.axis_index("c")
    s = lax.axis_index("s")
    wid = c * NS + s
    pltpu.sync_copy(hh_hbm.at[pl.ds(s * RB, RB)], acc.at[pl.ds(s * RB, RB)])

    @pl.when(s == 0)
    def _():
        pltpu.sync_copy(hh_hbm.at[pl.ds(TAIL_BASE, TAIL)],
                        acc.at[pl.ds(TAIL_BASE, TAIL)])

    plsc.subcore_barrier()

    def gstart(j, buf, sem):
        pltpu.make_async_copy(hh_hbm.at[sflat.at[pl.ds(j * CH, CH)]],
                              buf, sem).start()

    def gwait(j, buf, sem):
        pltpu.make_async_copy(hh_hbm.at[sflat.at[pl.ds(j * CH, CH)]],
                              buf, sem).wait()

    @pl.loop(0, NPHASE)
    def _(p):
        pltpu.sync_copy(src_hbm.at[pl.ds(wid * EPW + p * PHE, PHE)], sflat)
        pltpu.sync_copy(dst_hbm.at[pl.ds(wid * EPW + p * PHE, PHE)], dflat)
        _repack(dflat, dstv)

        bufs = (buf0, buf1, buf2, buf3)
        sems = (sem0, sem1, sem2, sem3)
        for k in range(3):
            gstart(k, bufs[k], sems[k])

        @pl.loop(0, PCH - 1, step=4)
        def _(j):
            for k in range(4):
                cidx = j + k

                @pl.when(cidx + 3 < PCH)
                def _():
                    gstart(cidx + 3, bufs[(k + 3) % 4], sems[(k + 3) % 4])

                gwait(cidx, bufs[k], sems[k])
                pltpu.sync_copy(bufs[k], acc.at[dstv.at[cidx]], add=True)

        gwait(PCH - 1, buf0, sem0)
        pltpu.sync_copy(buf0, acc.at[dstv.at[PCH - 1]], add=True)

    plsc.subcore_barrier()
    pltpu.sync_copy(acc.at[pl.ds(s * RB, RB)], out_hbm.at[c, pl.ds(s * RB, RB)])

    @pl.when(s == 0)
    def _():
        pltpu.sync_copy(acc.at[pl.ds(TAIL_BASE, TAIL)],
                        out_hbm.at[c, pl.ds(TAIL_BASE, TAIL)])


def _nrsqrt(d):
    i = lax.bitcast_convert_type(d, jnp.int32)
    i = jnp.int32(0x5F3759DF) - (i >> 1)
    y = lax.bitcast_convert_type(i, jnp.float32)
    for _ in range(3):
        y = y * (1.5 - 0.5 * d * y * y)
    return y


def _scale_body(x_ref, w_ref, deg_ref, hh_ref, dinv_ref):
    deg = deg_ref[0, :, 0:1] + deg_ref[1, :, 0:1] - 1.0
    dinv = _nrsqrt(deg)
    dinv_ref[...] = jnp.broadcast_to(dinv, (BR, 16))
    xw = jnp.dot(x_ref[...], w_ref[...], preferred_element_type=jnp.float32)
    hh_ref[...] = xw * dinv


def _tc_scale(x, w, degp):
    return pl.pallas_call(
        _scale_body,
        grid=(GRID,),
        in_specs=[
            pl.BlockSpec((BR, D), lambda i: (i, 0)),
            pl.BlockSpec((D, H), lambda i: (0, 0)),
            pl.BlockSpec((NC, BR, 16), lambda i: (0, i, 0)),
        ],
        out_specs=[
            pl.BlockSpec((BR, H), lambda i: (i, 0)),
            pl.BlockSpec((BR, 16), lambda i: (i, 0)),
        ],
        out_shape=[
            jax.ShapeDtypeStruct((N, H), jnp.float32),
            jax.ShapeDtypeStruct((N, 16), jnp.float32),
        ],
    )(x, w, degp)


def _post_body(p_ref, hh_ref, dinv_ref, b_ref, g_ref, bb_ref, w_ref, o_ref):
    dinv = dinv_ref[:, 0:1]
    y = (p_ref[0] + p_ref[1] - hh_ref[...]) * dinv + b_ref[...]
    t = jnp.maximum(y * INVS * g_ref[...] + bb_ref[...], 0.0)
    o_ref[...] = jnp.dot(t, w_ref[...], preferred_element_type=jnp.float32) * dinv


def _tc_post(p, hh, dinv, b, g, bb, w_next):
    return pl.pallas_call(
        _post_body,
        grid=(GRID,),
        in_specs=[
            pl.BlockSpec((NC, BR, H), lambda i: (0, i, 0)),
            pl.BlockSpec((BR, H), lambda i: (i, 0)),
            pl.BlockSpec((BR, 16), lambda i: (i, 0)),
            pl.BlockSpec((1, H), lambda i: (0, 0)),
            pl.BlockSpec((1, H), lambda i: (0, 0)),
            pl.BlockSpec((1, H), lambda i: (0, 0)),
            pl.BlockSpec((H, H), lambda i: (0, 0)),
        ],
        out_specs=pl.BlockSpec((BR, H), lambda i: (i, 0)),
        out_shape=jax.ShapeDtypeStruct((N, H), jnp.float32),
    )(p, hh, dinv, b, g, bb, w_next)


def _final_body(p_ref, hh_ref, dinv_ref, b_ref, g_ref, bb_ref,
                w1_ref, b1_ref, w2_ref, b2_ref, o_ref):
    dinv = dinv_ref[:, 0:1]
    y = (p_ref[0] + p_ref[1] - hh_ref[...]) * dinv + b_ref[...]
    t = jnp.maximum(y * INVS * g_ref[...] + bb_ref[...], 0.0)
    z = jnp.maximum(
        jnp.dot(t, w1_ref[...], preferred_element_type=jnp.float32) + b1_ref[...], 0.0)
    o_ref[...] = jnp.dot(z, w2_ref[...], preferred_element_type=jnp.float32) + b2_ref[...]


def _tc_final(p, hh, dinv, b, g, bb, w1, b1, w2, b2):
    return pl.pallas_call(
        _final_body,
        grid=(GRID,),
        in_specs=[
            pl.BlockSpec((NC, BR, H), lambda i: (0, i, 0)),
            pl.BlockSpec((BR, H), lambda i: (i, 0)),
            pl.BlockSpec((BR, 16), lambda i: (i, 0)),
            pl.BlockSpec((1, H), lambda i: (0, 0)),
            pl.BlockSpec((1, H), lambda i: (0, 0)),
            pl.BlockSpec((1, H), lambda i: (0, 0)),
            pl.BlockSpec((H, H // 2), lambda i: (0, 0)),
            pl.BlockSpec((1, H // 2), lambda i: (0, 0)),
            pl.BlockSpec((H // 2, 2), lambda i: (0, 0)),
            pl.BlockSpec((1, 2), lambda i: (0, 0)),
        ],
        out_specs=pl.BlockSpec((BR, 2), lambda i: (i, 0)),
        out_shape=jax.ShapeDtypeStruct((N, 2), jnp.float32),
    )(p, hh, dinv, b, g, bb, w1, b1, w2, b2)


def kernel(x, edge_index, batch, conv_w0, conv_b0, bn_g0, bn_b0,
           conv_w1, conv_b1, bn_g1, bn_b1, conv_w2, conv_b2, bn_g2, bn_b2,
           out_w1, out_b1, out_w2, out_b2):
    src_f = edge_index[0]
    dst_f = edge_index[1]
    ones16 = jnp.ones((N, 16), jnp.float32)

    degp = _sc_degree(dst_f, ones16)
    hh, dinv = _tc_scale(x, conv_w0, degp)

    p = _sc_gather_scatter(hh, src_f, dst_f)
    hh = _tc_post(p, hh, dinv, conv_b0.reshape(1, H), bn_g0.reshape(1, H),
                  bn_b0.reshape(1, H), conv_w1)

    p = _sc_gather_scatter(hh, src_f, dst_f)
    hh = _tc_post(p, hh, dinv, conv_b1.reshape(1, H), bn_g1.reshape(1, H),
                  bn_b1.reshape(1, H), conv_w2)

    p = _sc_gather_scatter(hh, src_f, dst_f)
    return _tc_final(p, hh, dinv, conv_b2.reshape(1, H), bn_g2.reshape(1, H),
                     bn_b2.reshape(1, H), out_w1, out_b1.reshape(1, H // 2),
                     out_w2, out_b2.reshape(1, 2))


# speedup vs baseline: 28.0590x; 1.0059x over previous
import functools

import jax
import jax.numpy as jnp
from jax import lax
from jax.experimental import pallas as pl
from jax.experimental.pallas import tpu as pltpu
from jax.experimental.pallas import tpu_sc as plsc

N = 10000
D = 128
H = 128
E = 320000
NC = 2
NS = 16
NW = NC * NS
EPW = E // NW
CH = 80
NCHUNK = EPW // CH
PCH = 25
PHE = PCH * CH
NPHASE = NCHUNK // PCH
RB = 624
TAIL_BASE = NS * RB
TAIL = N - TAIL_BASE
BR = 2000
GRID = N // BR
INVS = (1.0 + 1e-5) ** -0.5
MM_PREC = lax.Precision.DEFAULT

_mesh = plsc.VectorSubcoreMesh(core_axis_name="c", subcore_axis_name="s")


def _repack(flat, rows2d):
    @pl.loop(0, PCH)
    def _(r):
        @pl.loop(0, CH // 16)
        def _(g):
            rows2d[r, pl.ds(g * 16, 16)] = flat[pl.ds(r * CH + g * 16, 16)]


@functools.partial(
    pl.kernel,
    out_type=jax.ShapeDtypeStruct((NC, N, 16), jnp.float32),
    mesh=_mesh,
    scratch_types=[
        pltpu.VMEM((EPW,), jnp.int32),
        pltpu.VMEM((NCHUNK, CH), jnp.int32),
        pltpu.VMEM((CH, 16), jnp.float32),
        pltpu.VMEM_SHARED((N, 16), jnp.float32),
        pltpu.SemaphoreType.DMA,
    ],
)
def _sc_degree(dst_hbm, ones_hbm, out_hbm, dflat, dstv, onesb, acc, sem):
    c = lax.axis_index("c")
    s = lax.axis_index("s")
    wid = c * NS + s

    @pl.loop(0, CH)
    def _(i):
        onesb[i] = jnp.ones((16,), jnp.float32)

    pltpu.sync_copy(ones_hbm.at[pl.ds(s * RB, RB)], acc.at[pl.ds(s * RB, RB)])

    @pl.when(s == 0)
    def _():
        pltpu.sync_copy(ones_hbm.at[pl.ds(TAIL_BASE, TAIL)],
                        acc.at[pl.ds(TAIL_BASE, TAIL)])

    plsc.subcore_barrier()

    pltpu.sync_copy(dst_hbm.at[pl.ds(wid * EPW, EPW)], dflat)

    @pl.loop(0, NCHUNK)
    def _(r):
        @pl.loop(0, CH // 16)
        def _(g):
            dstv[r, pl.ds(g * 16, 16)] = dflat[pl.ds(r * CH + g * 16, 16)]

    @pl.loop(0, NCHUNK)
    def _(j):
        pltpu.async_copy(onesb, acc.at[dstv.at[j]], sem, add=True)

    @pl.loop(0, NCHUNK)
    def _(j):
        pltpu.make_async_copy(onesb, acc.at[dstv.at[0]], sem).wait()

    plsc.subcore_barrier()
    pltpu.sync_copy(acc.at[pl.ds(s * RB, RB)], out_hbm.at[c, pl.ds(s * RB, RB)])

    @pl.when(s == 0)
    def _():
        pltpu.sync_copy(acc.at[pl.ds(TAIL_BASE, TAIL)],
                        out_hbm.at[c, pl.ds(TAIL_BASE, TAIL)])


@functools.partial(
    pl.kernel,
    out_type=jax.ShapeDtypeStruct((NC, N, H), jnp.float32),
    mesh=_mesh,
    scratch_types=[
        pltpu.VMEM((PHE,), jnp.int32),
        pltpu.VMEM((PHE,), jnp.int32),
        pltpu.VMEM((PCH, CH), jnp.int32),
        pltpu.VMEM((CH, H), jnp.float32),
        pltpu.VMEM((CH, H), jnp.float32),
        pltpu.VMEM((CH, H), jnp.float32),
        pltpu.VMEM((CH, H), jnp.float32),
        pltpu.VMEM_SHARED((N, H), jnp.float32),
        pltpu.SemaphoreType.DMA,
        pltpu.SemaphoreType.DMA,
        pltpu.SemaphoreType.DMA,
        pltpu.SemaphoreType.DMA,
    ],
)
def _sc_gather_scatter(hh_hbm, src_hbm, dst_hbm, out_hbm, sflat, dflat, dstv,
                       buf0, buf1, buf2, buf3, acc,
                       sem0, sem1, sem2, sem3):
    c = lax.axis_index("c")
    s = lax.axis_index("s")
    wid = c * NS + s
    pltpu.sync_copy(hh_hbm.at[pl.ds(s * RB, RB)], acc.at[pl.ds(s * RB, RB)])

    @pl.when(s == 0)
    def _():
        pltpu.sync_copy(hh_hbm.at[pl.ds(TAIL_BASE, TAIL)],
                        acc.at[pl.ds(TAIL_BASE, TAIL)])

    plsc.subcore_barrier()

    def gstart(j, buf, sem):
        pltpu.make_async_copy(hh_hbm.at[sflat.at[pl.ds(j * CH, CH)]],
                              buf, sem).start()

    def gwait(j, buf, sem):
        pltpu.make_async_copy(hh_hbm.at[sflat.at[pl.ds(j * CH, CH)]],
                              buf, sem).wait()

    @pl.loop(0, NPHASE)
    def _(p):
        pltpu.sync_copy(src_hbm.at[pl.ds(wid * EPW + p * PHE, PHE)], sflat)
        pltpu.sync_copy(dst_hbm.at[pl.ds(wid * EPW + p * PHE, PHE)], dflat)
        _repack(dflat, dstv)

        bufs = (buf0, buf1, buf2, buf3)
        sems = (sem0, sem1, sem2, sem3)
        for k in range(3):
            gstart(k, bufs[k], sems[k])

        @pl.loop(0, PCH - 1, step=4)
        def _(j):
            for k in range(4):
                cidx = j + k

                @pl.when(cidx + 3 < PCH)
                def _():
                    gstart(cidx + 3, bufs[(k + 3) % 4], sems[(k + 3) % 4])

                gwait(cidx, bufs[k], sems[k])
                pltpu.sync_copy(bufs[k], acc.at[dstv.at[cidx]], add=True)

        gwait(PCH - 1, buf0, sem0)
        pltpu.sync_copy(buf0, acc.at[dstv.at[PCH - 1]], add=True)

    plsc.subcore_barrier()
    pltpu.sync_copy(acc.at[pl.ds(s * RB, RB)], out_hbm.at[c, pl.ds(s * RB, RB)])

    @pl.when(s == 0)
    def _():
        pltpu.sync_copy(acc.at[pl.ds(TAIL_BASE, TAIL)],
                        out_hbm.at[c, pl.ds(TAIL_BASE, TAIL)])


def _nrsqrt(d):
    i = lax.bitcast_convert_type(d, jnp.int32)
    i = jnp.int32(0x5F3759DF) - (i >> 1)
    y = lax.bitcast_convert_type(i, jnp.float32)
    for _ in range(3):
        y = y * (1.5 - 0.5 * d * y * y)
    return y


def _scale_body(x_ref, w_ref, deg_ref, hh_ref, dinv_ref):
    deg = deg_ref[0, :, 0:1] + deg_ref[1, :, 0:1] - 1.0
    dinv = _nrsqrt(deg)
    dinv_ref[...] = jnp.broadcast_to(dinv, (BR, 16))
    xw = jnp.dot(x_ref[...], w_ref[...], preferred_element_type=jnp.float32, precision=MM_PREC)
    hh_ref[...] = xw * dinv


def _tc_scale(x, w, degp):
    return pl.pallas_call(
        _scale_body,
        grid=(GRID,),
        in_specs=[
            pl.BlockSpec((BR, D), lambda i: (i, 0)),
            pl.BlockSpec((D, H), lambda i: (0, 0)),
            pl.BlockSpec((NC, BR, 16), lambda i: (0, i, 0)),
        ],
        out_specs=[
            pl.BlockSpec((BR, H), lambda i: (i, 0)),
            pl.BlockSpec((BR, 16), lambda i: (i, 0)),
        ],
        out_shape=[
            jax.ShapeDtypeStruct((N, H), jnp.float32),
            jax.ShapeDtypeStruct((N, 16), jnp.float32),
        ],
    )(x, w, degp)


def _post_body(p_ref, hh_ref, dinv_ref, b_ref, g_ref, bb_ref, w_ref, o_ref):
    dinv = dinv_ref[:, 0:1]
    y = (p_ref[0] + p_ref[1] - hh_ref[...]) * dinv + b_ref[...]
    t = jnp.maximum(y * INVS * g_ref[...] + bb_ref[...], 0.0)
    o_ref[...] = jnp.dot(t, w_ref[...], preferred_element_type=jnp.float32, precision=MM_PREC) * dinv


def _tc_post(p, hh, dinv, b, g, bb, w_next):
    return pl.pallas_call(
        _post_body,
        grid=(GRID,),
        in_specs=[
            pl.BlockSpec((NC, BR, H), lambda i: (0, i, 0)),
            pl.BlockSpec((BR, H), lambda i: (i, 0)),
            pl.BlockSpec((BR, 16), lambda i: (i, 0)),
            pl.BlockSpec((1, H), lambda i: (0, 0)),
            pl.BlockSpec((1, H), lambda i: (0, 0)),
            pl.BlockSpec((1, H), lambda i: (0, 0)),
            pl.BlockSpec((H, H), lambda i: (0, 0)),
        ],
        out_specs=pl.BlockSpec((BR, H), lambda i: (i, 0)),
        out_shape=jax.ShapeDtypeStruct((N, H), jnp.float32),
    )(p, hh, dinv, b, g, bb, w_next)


def _final_body(p_ref, hh_ref, dinv_ref, b_ref, g_ref, bb_ref,
                w1_ref, b1_ref, w2_ref, b2_ref, o_ref):
    dinv = dinv_ref[:, 0:1]
    y = (p_ref[0] + p_ref[1] - hh_ref[...]) * dinv + b_ref[...]
    t = jnp.maximum(y * INVS * g_ref[...] + bb_ref[...], 0.0)
    z = jnp.maximum(
        jnp.dot(t, w1_ref[...], preferred_element_type=jnp.float32, precision=MM_PREC) + b1_ref[...], 0.0)
    o_ref[...] = jnp.dot(z, w2_ref[...], preferred_element_type=jnp.float32, precision=MM_PREC) + b2_ref[...]


def _tc_final(p, hh, dinv, b, g, bb, w1, b1, w2, b2):
    return pl.pallas_call(
        _final_body,
        grid=(GRID,),
        in_specs=[
            pl.BlockSpec((NC, BR, H), lambda i: (0, i, 0)),
            pl.BlockSpec((BR, H), lambda i: (i, 0)),
            pl.BlockSpec((BR, 16), lambda i: (i, 0)),
            pl.BlockSpec((1, H), lambda i: (0, 0)),
            pl.BlockSpec((1, H), lambda i: (0, 0)),
            pl.BlockSpec((1, H), lambda i: (0, 0)),
            pl.BlockSpec((H, H // 2), lambda i: (0, 0)),
            pl.BlockSpec((1, H // 2), lambda i: (0, 0)),
            pl.BlockSpec((H // 2, 2), lambda i: (0, 0)),
            pl.BlockSpec((1, 2), lambda i: (0, 0)),
        ],
        out_specs=pl.BlockSpec((BR, 2), lambda i: (i, 0)),
        out_shape=jax.ShapeDtypeStruct((N, 2), jnp.float32),
    )(p, hh, dinv, b, g, bb, w1, b1, w2, b2)


def kernel(x, edge_index, batch, conv_w0, conv_b0, bn_g0, bn_b0,
           conv_w1, conv_b1, bn_g1, bn_b1, conv_w2, conv_b2, bn_g2, bn_b2,
           out_w1, out_b1, out_w2, out_b2):
    src_f = edge_index[0]
    dst_f = edge_index[1]
    ones16 = jnp.ones((N, 16), jnp.float32)

    degp = _sc_degree(dst_f, ones16)
    hh, dinv = _tc_scale(x, conv_w0, degp)

    p = _sc_gather_scatter(hh, src_f, dst_f)
    hh = _tc_post(p, hh, dinv, conv_b0.reshape(1, H), bn_g0.reshape(1, H),
                  bn_b0.reshape(1, H), conv_w1)

    p = _sc_gather_scatter(hh, src_f, dst_f)
    hh = _tc_post(p, hh, dinv, conv_b1.reshape(1, H), bn_g1.reshape(1, H),
                  bn_b1.reshape(1, H), conv_w2)

    p = _sc_gather_scatter(hh, src_f, dst_f)
    return _tc_final(p, hh, dinv, conv_b2.reshape(1, H), bn_g2.reshape(1, H),
                     bn_b2.reshape(1, H), out_w1, out_b1.reshape(1, H // 2),
                     out_w2, out_b2.reshape(1, 2))


# speedup vs baseline: 28.9519x; 1.0318x over previous
import functools

import jax
import jax.numpy as jnp
from jax import lax
from jax.experimental import pallas as pl
from jax.experimental.pallas import tpu as pltpu
from jax.experimental.pallas import tpu_sc as plsc

N = 10000
D = 128
H = 128
E = 320000
NC = 2
NS = 16
NW = NC * NS
EPW = E // NW
CH = 80
NCHUNK = EPW // CH
PCH = 25
PHE = PCH * CH
NPHASE = NCHUNK // PCH
RB = 624
TAIL_BASE = NS * RB
TAIL = N - TAIL_BASE
BR = 2000
GRID = N // BR
INVS = (1.0 + 1e-5) ** -0.5
MM_PREC = lax.Precision.DEFAULT

_mesh = plsc.VectorSubcoreMesh(core_axis_name="c", subcore_axis_name="s")


def _repack(flat, rows2d):
    @pl.loop(0, PCH)
    def _(r):
        @pl.loop(0, CH // 16)
        def _(g):
            rows2d[r, pl.ds(g * 16, 16)] = flat[pl.ds(r * CH + g * 16, 16)]


@functools.partial(
    pl.kernel,
    out_type=jax.ShapeDtypeStruct((NC, N, 16), jnp.float32),
    mesh=_mesh,
    scratch_types=[
        pltpu.VMEM((EPW,), jnp.int32),
        pltpu.VMEM((NCHUNK, CH), jnp.int32),
        pltpu.VMEM((CH, 16), jnp.float32),
        pltpu.VMEM_SHARED((N, 16), jnp.float32),
        pltpu.SemaphoreType.DMA,
    ],
)
def _sc_degree(dst_hbm, ones_hbm, out_hbm, dflat, dstv, onesb, acc, sem):
    c = lax.axis_index("c")
    s = lax.axis_index("s")
    wid = c * NS + s

    @pl.loop(0, CH)
    def _(i):
        onesb[i] = jnp.ones((16,), jnp.float32)

    pltpu.sync_copy(ones_hbm.at[pl.ds(s * RB, RB)], acc.at[pl.ds(s * RB, RB)])

    @pl.when(s == 0)
    def _():
        pltpu.sync_copy(ones_hbm.at[pl.ds(TAIL_BASE, TAIL)],
                        acc.at[pl.ds(TAIL_BASE, TAIL)])

    plsc.subcore_barrier()

    pltpu.sync_copy(dst_hbm.at[pl.ds(wid * EPW, EPW)], dflat)

    @pl.loop(0, NCHUNK)
    def _(r):
        @pl.loop(0, CH // 16)
        def _(g):
            dstv[r, pl.ds(g * 16, 16)] = dflat[pl.ds(r * CH + g * 16, 16)]

    @pl.loop(0, NCHUNK)
    def _(j):
        pltpu.async_copy(onesb, acc.at[dstv.at[j]], sem, add=True)

    @pl.loop(0, NCHUNK)
    def _(j):
        pltpu.make_async_copy(onesb, acc.at[dstv.at[0]], sem).wait()

    plsc.subcore_barrier()
    pltpu.sync_copy(acc.at[pl.ds(s * RB, RB)], out_hbm.at[c, pl.ds(s * RB, RB)])

    @pl.when(s == 0)
    def _():
        pltpu.sync_copy(acc.at[pl.ds(TAIL_BASE, TAIL)],
                        out_hbm.at[c, pl.ds(TAIL_BASE, TAIL)])


@functools.partial(
    pl.kernel,
    out_type=jax.ShapeDtypeStruct((NC, N, H), jnp.float32),
    mesh=_mesh,
    scratch_types=[
        pltpu.VMEM((PHE,), jnp.int32),
        pltpu.VMEM((PHE,), jnp.int32),
        pltpu.VMEM((PCH, CH), jnp.int32),
        pltpu.VMEM((CH, H), jnp.float32),
        pltpu.VMEM((CH, H), jnp.float32),
        pltpu.VMEM((CH, H), jnp.float32),
        pltpu.VMEM((CH, H), jnp.float32),
        pltpu.VMEM_SHARED((N, H), jnp.float32),
        pltpu.SemaphoreType.DMA,
        pltpu.SemaphoreType.DMA,
        pltpu.SemaphoreType.DMA,
        pltpu.SemaphoreType.DMA,
    ],
)
def _sc_gather_scatter(hh_hbm, src_hbm, dst_hbm, out_hbm, sflat, dflat, dstv,
                       buf0, buf1, buf2, buf3, acc,
                       sem0, sem1, sem2, sem3):
    c = lax.axis_index("c")
    s = lax.axis_index("s")
    wid = c * NS + s
    pltpu.sync_copy(hh_hbm.at[pl.ds(s * RB, RB)], acc.at[pl.ds(s * RB, RB)])

    @pl.when(s == 0)
    def _():
        pltpu.sync_copy(hh_hbm.at[pl.ds(TAIL_BASE, TAIL)],
                        acc.at[pl.ds(TAIL_BASE, TAIL)])

    plsc.subcore_barrier()

    def gstart(j, buf, sem):
        pltpu.make_async_copy(hh_hbm.at[sflat.at[pl.ds(j * CH, CH)]],
                              buf, sem).start()

    def gwait(j, buf, sem):
        pltpu.make_async_copy(hh_hbm.at[sflat.at[pl.ds(j * CH, CH)]],
                              buf, sem).wait()

    @pl.loop(0, NPHASE)
    def _(p):
        pltpu.sync_copy(src_hbm.at[pl.ds(wid * EPW + p * PHE, PHE)], sflat)
        pltpu.sync_copy(dst_hbm.at[pl.ds(wid * EPW + p * PHE, PHE)], dflat)
        _repack(dflat, dstv)

        bufs = (buf0, buf1, buf2, buf3)
        sems = (sem0, sem1, sem2, sem3)
        for k in range(3):
            gstart(k, bufs[k], sems[k])

        @pl.loop(0, PCH - 1, step=4)
        def _(j):
            for k in range(4):
                cidx = j + k

                @pl.when(cidx + 3 < PCH)
                def _():
                    gstart(cidx + 3, bufs[(k + 3) % 4], sems[(k + 3) % 4])

                gwait(cidx, bufs[k], sems[k])
                pltpu.sync_copy(bufs[k], acc.at[dstv.at[cidx]], add=True)

        gwait(PCH - 1, buf0, sem0)
        pltpu.sync_copy(buf0, acc.at[dstv.at[PCH - 1]], add=True)

    plsc.subcore_barrier()
    pltpu.sync_copy(acc.at[pl.ds(s * RB, RB)], out_hbm.at[c, pl.ds(s * RB, RB)])

    @pl.when(s == 0)
    def _():
        pltpu.sync_copy(acc.at[pl.ds(TAIL_BASE, TAIL)],
                        out_hbm.at[c, pl.ds(TAIL_BASE, TAIL)])


def _split_body(ei_ref, s_ref, d_ref):
    s_ref[...] = ei_ref[0]
    d_ref[...] = ei_ref[1]


def _tc_split(edge_index):
    return pl.pallas_call(
        _split_body,
        grid=(1,),
        in_specs=[pl.BlockSpec((2, E), lambda i: (0, 0))],
        out_specs=[
            pl.BlockSpec((E,), lambda i: (0,)),
            pl.BlockSpec((E,), lambda i: (0,)),
        ],
        out_shape=[
            jax.ShapeDtypeStruct((E,), jnp.int32),
            jax.ShapeDtypeStruct((E,), jnp.int32),
        ],
    )(edge_index)

def _nrsqrt(d):
    i = lax.bitcast_convert_type(d, jnp.int32)
    i = jnp.int32(0x5F3759DF) - (i >> 1)
    y = lax.bitcast_convert_type(i, jnp.float32)
    for _ in range(3):
        y = y * (1.5 - 0.5 * d * y * y)
    return y


def _scale_body(x_ref, w_ref, deg_ref, hh_ref, dinv_ref):
    deg = deg_ref[0, :, 0:1] + deg_ref[1, :, 0:1] - 1.0
    dinv = _nrsqrt(deg)
    dinv_ref[...] = jnp.broadcast_to(dinv, (BR, 16))
    xw = jnp.dot(x_ref[...], w_ref[...], preferred_element_type=jnp.float32, precision=MM_PREC)
    hh_ref[...] = xw * dinv


def _tc_scale(x, w, degp):
    return pl.pallas_call(
        _scale_body,
        grid=(GRID,),
        in_specs=[
            pl.BlockSpec((BR, D), lambda i: (i, 0)),
            pl.BlockSpec((D, H), lambda i: (0, 0)),
            pl.BlockSpec((NC, BR, 16), lambda i: (0, i, 0)),
        ],
        out_specs=[
            pl.BlockSpec((BR, H), lambda i: (i, 0)),
            pl.BlockSpec((BR, 16), lambda i: (i, 0)),
        ],
        out_shape=[
            jax.ShapeDtypeStruct((N, H), jnp.float32),
            jax.ShapeDtypeStruct((N, 16), jnp.float32),
        ],
    )(x, w, degp)


def _post_body(p_ref, hh_ref, dinv_ref, b_ref, g_ref, bb_ref, w_ref, o_ref):
    dinv = dinv_ref[:, 0:1]
    y = (p_ref[0] + p_ref[1] - hh_ref[...]) * dinv + b_ref[...]
    t = jnp.maximum(y * INVS * g_ref[...] + bb_ref[...], 0.0)
    o_ref[...] = jnp.dot(t, w_ref[...], preferred_element_type=jnp.float32, precision=MM_PREC) * dinv


def _tc_post(p, hh, dinv, b, g, bb, w_next):
    return pl.pallas_call(
        _post_body,
        grid=(GRID,),
        in_specs=[
            pl.BlockSpec((NC, BR, H), lambda i: (0, i, 0)),
            pl.BlockSpec((BR, H), lambda i: (i, 0)),
            pl.BlockSpec((BR, 16), lambda i: (i, 0)),
            pl.BlockSpec((1, H), lambda i: (0, 0)),
            pl.BlockSpec((1, H), lambda i: (0, 0)),
            pl.BlockSpec((1, H), lambda i: (0, 0)),
            pl.BlockSpec((H, H), lambda i: (0, 0)),
        ],
        out_specs=pl.BlockSpec((BR, H), lambda i: (i, 0)),
        out_shape=jax.ShapeDtypeStruct((N, H), jnp.float32),
    )(p, hh, dinv, b, g, bb, w_next)


def _final_body(p_ref, hh_ref, dinv_ref, b_ref, g_ref, bb_ref,
                w1_ref, b1_ref, w2_ref, b2_ref, o_ref):
    dinv = dinv_ref[:, 0:1]
    y = (p_ref[0] + p_ref[1] - hh_ref[...]) * dinv + b_ref[...]
    t = jnp.maximum(y * INVS * g_ref[...] + bb_ref[...], 0.0)
    z = jnp.maximum(
        jnp.dot(t, w1_ref[...], preferred_element_type=jnp.float32, precision=MM_PREC) + b1_ref[...], 0.0)
    o_ref[...] = jnp.dot(z, w2_ref[...], preferred_element_type=jnp.float32, precision=MM_PREC) + b2_ref[...]


def _tc_final(p, hh, dinv, b, g, bb, w1, b1, w2, b2):
    return pl.pallas_call(
        _final_body,
        grid=(GRID,),
        in_specs=[
            pl.BlockSpec((NC, BR, H), lambda i: (0, i, 0)),
            pl.BlockSpec((BR, H), lambda i: (i, 0)),
            pl.BlockSpec((BR, 16), lambda i: (i, 0)),
            pl.BlockSpec((1, H), lambda i: (0, 0)),
            pl.BlockSpec((1, H), lambda i: (0, 0)),
            pl.BlockSpec((1, H), lambda i: (0, 0)),
            pl.BlockSpec((H, H // 2), lambda i: (0, 0)),
            pl.BlockSpec((1, H // 2), lambda i: (0, 0)),
            pl.BlockSpec((H // 2, 2), lambda i: (0, 0)),
            pl.BlockSpec((1, 2), lambda i: (0, 0)),
        ],
        out_specs=pl.BlockSpec((BR, 2), lambda i: (i, 0)),
        out_shape=jax.ShapeDtypeStruct((N, 2), jnp.float32),
    )(p, hh, dinv, b, g, bb, w1, b1, w2, b2)


def kernel(x, edge_index, batch, conv_w0, conv_b0, bn_g0, bn_b0,
           conv_w1, conv_b1, bn_g1, bn_b1, conv_w2, conv_b2, bn_g2, bn_b2,
           out_w1, out_b1, out_w2, out_b2):
    src_f, dst_f = _tc_split(edge_index)
    ones16 = jnp.ones((N, 16), jnp.float32)

    degp = _sc_degree(dst_f, ones16)
    hh, dinv = _tc_scale(x, conv_w0, degp)

    p = _sc_gather_scatter(hh, src_f, dst_f)
    hh = _tc_post(p, hh, dinv, conv_b0.reshape(1, H), bn_g0.reshape(1, H),
                  bn_b0.reshape(1, H), conv_w1)

    p = _sc_gather_scatter(hh, src_f, dst_f)
    hh = _tc_post(p, hh, dinv, conv_b1.reshape(1, H), bn_g1.reshape(1, H),
                  bn_b1.reshape(1, H), conv_w2)

    p = _sc_gather_scatter(hh, src_f, dst_f)
    return _tc_final(p, hh, dinv, conv_b2.reshape(1, H), bn_g2.reshape(1, H),
                     bn_b2.reshape(1, H), out_w1, out_b1.reshape(1, H // 2),
                     out_w2, out_b2.reshape(1, 2))


# speedup vs baseline: 29.0494x; 1.0034x over previous
import functools

import jax
import jax.numpy as jnp
from jax import lax
from jax.experimental import pallas as pl
from jax.experimental.pallas import tpu as pltpu
from jax.experimental.pallas import tpu_sc as plsc

N = 10000
D = 128
H = 128
E = 320000
NC = 2
NS = 16
NW = NC * NS
EPW = E // NW
CH = 80
NCHUNK = EPW // CH
PCH = 25
PHE = PCH * CH
NPHASE = NCHUNK // PCH
RB = 624
TAIL_BASE = NS * RB
TAIL = N - TAIL_BASE
BR = 2000
GRID = N // BR
INVS = (1.0 + 1e-5) ** -0.5
MM_PREC = lax.Precision.DEFAULT

_mesh = plsc.VectorSubcoreMesh(core_axis_name="c", subcore_axis_name="s")


def _repack(flat, rows2d):
    @pl.loop(0, PCH)
    def _(r):
        @pl.loop(0, CH // 16)
        def _(g):
            rows2d[r, pl.ds(g * 16, 16)] = flat[pl.ds(r * CH + g * 16, 16)]


@functools.partial(
    pl.kernel,
    out_type=jax.ShapeDtypeStruct((NC, N, 16), jnp.float32),
    mesh=_mesh,
    scratch_types=[
        pltpu.VMEM((EPW,), jnp.int32),
        pltpu.VMEM((NCHUNK, CH), jnp.int32),
        pltpu.VMEM((CH, 16), jnp.float32),
        pltpu.VMEM_SHARED((N, 16), jnp.float32),
        pltpu.SemaphoreType.DMA,
    ],
)
def _sc_degree(dst_hbm, ones_hbm, out_hbm, dflat, dstv, onesb, acc, sem):
    c = lax.axis_index("c")
    s = lax.axis_index("s")
    wid = c * NS + s

    @pl.loop(0, CH)
    def _(i):
        onesb[i] = jnp.ones((16,), jnp.float32)

    pltpu.sync_copy(ones_hbm.at[pl.ds(s * RB, RB)], acc.at[pl.ds(s * RB, RB)])

    @pl.when(s == 0)
    def _():
        pltpu.sync_copy(ones_hbm.at[pl.ds(TAIL_BASE, TAIL)],
                        acc.at[pl.ds(TAIL_BASE, TAIL)])

    plsc.subcore_barrier()

    pltpu.sync_copy(dst_hbm.at[pl.ds(wid * EPW, EPW)], dflat)

    @pl.loop(0, NCHUNK)
    def _(r):
        @pl.loop(0, CH // 16)
        def _(g):
            dstv[r, pl.ds(g * 16, 16)] = dflat[pl.ds(r * CH + g * 16, 16)]

    @pl.loop(0, NCHUNK)
    def _(j):
        pltpu.async_copy(onesb, acc.at[dstv.at[j]], sem, add=True)

    @pl.loop(0, NCHUNK)
    def _(j):
        pltpu.make_async_copy(onesb, acc.at[dstv.at[0]], sem).wait()

    plsc.subcore_barrier()
    pltpu.sync_copy(acc.at[pl.ds(s * RB, RB)], out_hbm.at[c, pl.ds(s * RB, RB)])

    @pl.when(s == 0)
    def _():
        pltpu.sync_copy(acc.at[pl.ds(TAIL_BASE, TAIL)],
                        out_hbm.at[c, pl.ds(TAIL_BASE, TAIL)])


@functools.partial(
    pl.kernel,
    out_type=jax.ShapeDtypeStruct((NC, N, H), jnp.float32),
    mesh=_mesh,
    scratch_types=[
        pltpu.VMEM((PHE,), jnp.int32),
        pltpu.VMEM((PHE,), jnp.int32),
        pltpu.VMEM((PCH, CH), jnp.int32),
        pltpu.VMEM((CH, H), jnp.float32),
        pltpu.VMEM((CH, H), jnp.float32),
        pltpu.VMEM((CH, H), jnp.float32),
        pltpu.VMEM((CH, H), jnp.float32),
        pltpu.VMEM_SHARED((N, H), jnp.float32),
        pltpu.SemaphoreType.DMA,
        pltpu.SemaphoreType.DMA,
        pltpu.SemaphoreType.DMA,
        pltpu.SemaphoreType.DMA,
    ],
)
def _sc_gather_scatter(hh_hbm, src_hbm, dst_hbm, out_hbm, sflat, dflat, dstv,
                       buf0, buf1, buf2, buf3, acc,
                       sem0, sem1, sem2, sem3):
    c = lax.axis_index("c")
    s = lax.axis_index("s")
    wid = c * NS + s
    pltpu.sync_copy(hh_hbm.at[pl.ds(s * RB, RB)], acc.at[pl.ds(s * RB, RB)])

    @pl.when(s == 0)
    def _():
        pltpu.sync_copy(hh_hbm.at[pl.ds(TAIL_BASE, TAIL)],
                        acc.at[pl.ds(TAIL_BASE, TAIL)])

    plsc.subcore_barrier()

    def gstart(j, buf, sem):
        pltpu.make_async_copy(hh_hbm.at[sflat.at[pl.ds(j * CH, CH)]],
                              buf, sem).start()

    def gwait(j, buf, sem):
        pltpu.make_async_copy(hh_hbm.at[sflat.at[pl.ds(j * CH, CH)]],
                              buf, sem).wait()

    @pl.loop(0, NPHASE)
    def _(p):
        pltpu.sync_copy(src_hbm.at[pl.ds(wid * EPW + p * PHE, PHE)], sflat)
        pltpu.sync_copy(dst_hbm.at[pl.ds(wid * EPW + p * PHE, PHE)], dflat)
        _repack(dflat, dstv)

        bufs = (buf0, buf1, buf2, buf3)
        sems = (sem0, sem1, sem2, sem3)
        for k in range(3):
            gstart(k, bufs[k], sems[k])

        @pl.loop(0, PCH - 1, step=4)
        def _(j):
            for k in range(4):
                cidx = j + k

                @pl.when(cidx + 3 < PCH)
                def _():
                    gstart(cidx + 3, bufs[(k + 3) % 4], sems[(k + 3) % 4])

                gwait(cidx, bufs[k], sems[k])
                pltpu.sync_copy(bufs[k], acc.at[dstv.at[cidx]], add=True)

        gwait(PCH - 1, buf0, sem0)
        pltpu.sync_copy(buf0, acc.at[dstv.at[PCH - 1]], add=True)

    plsc.subcore_barrier()
    pltpu.sync_copy(acc.at[pl.ds(s * RB, RB)], out_hbm.at[c, pl.ds(s * RB, RB)])

    @pl.when(s == 0)
    def _():
        pltpu.sync_copy(acc.at[pl.ds(TAIL_BASE, TAIL)],
                        out_hbm.at[c, pl.ds(TAIL_BASE, TAIL)])


def _split_body(ei_ref, s_ref, d_ref, o_ref):
    s_ref[...] = ei_ref[0]
    d_ref[...] = ei_ref[1]
    o_ref[...] = jnp.ones((N, 16), jnp.float32)


def _tc_split(edge_index):
    return pl.pallas_call(
        _split_body,
        grid=(1,),
        in_specs=[pl.BlockSpec((2, E), lambda i: (0, 0))],
        out_specs=[
            pl.BlockSpec((E,), lambda i: (0,)),
            pl.BlockSpec((E,), lambda i: (0,)),
            pl.BlockSpec((N, 16), lambda i: (0, 0)),
        ],
        out_shape=[
            jax.ShapeDtypeStruct((E,), jnp.int32),
            jax.ShapeDtypeStruct((E,), jnp.int32),
            jax.ShapeDtypeStruct((N, 16), jnp.float32),
        ],
    )(edge_index)

def _nrsqrt(d):
    i = lax.bitcast_convert_type(d, jnp.int32)
    i = jnp.int32(0x5F3759DF) - (i >> 1)
    y = lax.bitcast_convert_type(i, jnp.float32)
    for _ in range(3):
        y = y * (1.5 - 0.5 * d * y * y)
    return y


def _scale_body(x_ref, w_ref, deg_ref, hh_ref, dinv_ref):
    deg = deg_ref[0, :, 0:1] + deg_ref[1, :, 0:1] - 1.0
    dinv = _nrsqrt(deg)
    dinv_ref[...] = jnp.broadcast_to(dinv, (BR, 16))
    xw = jnp.dot(x_ref[...], w_ref[...], preferred_element_type=jnp.float32, precision=MM_PREC)
    hh_ref[...] = xw * dinv


def _tc_scale(x, w, degp):
    return pl.pallas_call(
        _scale_body,
        grid=(GRID,),
        in_specs=[
            pl.BlockSpec((BR, D), lambda i: (i, 0)),
            pl.BlockSpec((D, H), lambda i: (0, 0)),
            pl.BlockSpec((NC, BR, 16), lambda i: (0, i, 0)),
        ],
        out_specs=[
            pl.BlockSpec((BR, H), lambda i: (i, 0)),
            pl.BlockSpec((BR, 16), lambda i: (i, 0)),
        ],
        out_shape=[
            jax.ShapeDtypeStruct((N, H), jnp.float32),
            jax.ShapeDtypeStruct((N, 16), jnp.float32),
        ],
    )(x, w, degp)


def _post_body(p_ref, hh_ref, dinv_ref, b_ref, g_ref, bb_ref, w_ref, o_ref):
    dinv = dinv_ref[:, 0:1]
    y = (p_ref[0] + p_ref[1] - hh_ref[...]) * dinv + b_ref[...]
    t = jnp.maximum(y * INVS * g_ref[...] + bb_ref[...], 0.0)
    o_ref[...] = jnp.dot(t, w_ref[...], preferred_element_type=jnp.float32, precision=MM_PREC) * dinv


def _tc_post(p, hh, dinv, b, g, bb, w_next):
    return pl.pallas_call(
        _post_body,
        grid=(GRID,),
        in_specs=[
            pl.BlockSpec((NC, BR, H), lambda i: (0, i, 0)),
            pl.BlockSpec((BR, H), lambda i: (i, 0)),
            pl.BlockSpec((BR, 16), lambda i: (i, 0)),
            pl.BlockSpec((1, H), lambda i: (0, 0)),
            pl.BlockSpec((1, H), lambda i: (0, 0)),
            pl.BlockSpec((1, H), lambda i: (0, 0)),
            pl.BlockSpec((H, H), lambda i: (0, 0)),
        ],
        out_specs=pl.BlockSpec((BR, H), lambda i: (i, 0)),
        out_shape=jax.ShapeDtypeStruct((N, H), jnp.float32),
    )(p, hh, dinv, b, g, bb, w_next)


def _final_body(p_ref, hh_ref, dinv_ref, b_ref, g_ref, bb_ref,
                w1_ref, b1_ref, w2_ref, b2_ref, o_ref):
    dinv = dinv_ref[:, 0:1]
    y = (p_ref[0] + p_ref[1] - hh_ref[...]) * dinv + b_ref[...]
    t = jnp.maximum(y * INVS * g_ref[...] + bb_ref[...], 0.0)
    z = jnp.maximum(
        jnp.dot(t, w1_ref[...], preferred_element_type=jnp.float32, precision=MM_PREC) + b1_ref[...], 0.0)
    o_ref[...] = jnp.dot(z, w2_ref[...], preferred_element_type=jnp.float32, precision=MM_PREC) + b2_ref[...]


def _tc_final(p, hh, dinv, b, g, bb, w1, b1, w2, b2):
    return pl.pallas_call(
        _final_body,
        grid=(GRID,),
        in_specs=[
            pl.BlockSpec((NC, BR, H), lambda i: (0, i, 0)),
            pl.BlockSpec((BR, H), lambda i: (i, 0)),
            pl.BlockSpec((BR, 16), lambda i: (i, 0)),
            pl.BlockSpec((1, H), lambda i: (0, 0)),
            pl.BlockSpec((1, H), lambda i: (0, 0)),
            pl.BlockSpec((1, H), lambda i: (0, 0)),
            pl.BlockSpec((H, H // 2), lambda i: (0, 0)),
            pl.BlockSpec((1, H // 2), lambda i: (0, 0)),
            pl.BlockSpec((H // 2, 2), lambda i: (0, 0)),
            pl.BlockSpec((1, 2), lambda i: (0, 0)),
        ],
        out_specs=pl.BlockSpec((BR, 2), lambda i: (i, 0)),
        out_shape=jax.ShapeDtypeStruct((N, 2), jnp.float32),
    )(p, hh, dinv, b, g, bb, w1, b1, w2, b2)


def kernel(x, edge_index, batch, conv_w0, conv_b0, bn_g0, bn_b0,
           conv_w1, conv_b1, bn_g1, bn_b1, conv_w2, conv_b2, bn_g2, bn_b2,
           out_w1, out_b1, out_w2, out_b2):
    src_f, dst_f, ones16 = _tc_split(edge_index)

    degp = _sc_degree(dst_f, ones16)
    hh, dinv = _tc_scale(x, conv_w0, degp)

    p = _sc_gather_scatter(hh, src_f, dst_f)
    hh = _tc_post(p, hh, dinv, conv_b0.reshape(1, H), bn_g0.reshape(1, H),
                  bn_b0.reshape(1, H), conv_w1)

    p = _sc_gather_scatter(hh, src_f, dst_f)
    hh = _tc_post(p, hh, dinv, conv_b1.reshape(1, H), bn_g1.reshape(1, H),
                  bn_b1.reshape(1, H), conv_w2)

    p = _sc_gather_scatter(hh, src_f, dst_f)
    return _tc_final(p, hh, dinv, conv_b2.reshape(1, H), bn_g2.reshape(1, H),
                     bn_b2.reshape(1, H), out_w1, out_b1.reshape(1, H // 2),
                     out_w2, out_b2.reshape(1, 2))
